# Initial kernel scaffold; baseline (speedup 1.0000x reference)
import jax
import jax.numpy as jnp
from jax.experimental import pallas as pl


def kernel(field, pos, bc_type, edge_index, face_normals, face_areas, face_type, params):
    raise NotImplementedError("write your pallas kernel here")


# speedup vs baseline: 4.4033x; 4.4033x over previous
import functools

import jax
import jax.numpy as jnp
from jax.experimental import pallas as pl
from jax.experimental.pallas import tpu as pltpu
from jax.experimental.pallas import tpu_sc as plsc

N = 10000
E = 160000
H = 128
NHEADS = 8
NPAD = 10240
NC, NS = 2, 16
NW = NC * NS
TT = 2000

_F32 = jnp.float32


def _ln(h, g, b):
    mu = jnp.mean(h, axis=1, keepdims=True)
    var = jnp.mean((h - mu) ** 2, axis=1, keepdims=True)
    return (h - mu) * jax.lax.rsqrt(var + 1e-5) * g + b


def _dot(a, b):
    return jnp.dot(a, b, preferred_element_type=_F32)


def _enc_body(xin, w1, b1, w2, b2, w3, b3, g, bt, out):
    h = jnp.maximum(_dot(xin[...], w1[...]) + b1[...], 0.0)
    h = jnp.maximum(_dot(h, w2[...]) + b2[...], 0.0)
    h = _dot(h, w3[...]) + b3[...]
    out[...] = _ln(h, g[...], bt[...])


def _mlp16(xin, w1, b1, w2, b2, w3, b3, g, bt):
    rows = xin.shape[0]
    grid = rows // TT
    wspec = lambda a: pl.BlockSpec(a.shape, lambda i: (0,) * a.ndim)
    return pl.pallas_call(
        _enc_body,
        grid=(grid,),
        in_specs=[pl.BlockSpec((TT, 16), lambda i: (i, 0))]
        + [wspec(a) for a in (w1, b1, w2, b2, w3, b3, g, bt)],
        out_specs=pl.BlockSpec((TT, H), lambda i: (i, 0)),
        out_shape=jax.ShapeDtypeStruct((rows, H), _F32),
    )(xin, w1, b1, w2, b2, w3, b3, g, bt)


def _edge_body(gxd, gxs, e, w1, b1, w2, b2, w3, b3, g, bt, wqk, wv, sel, exp16,
               e1_out, u_out, p_out):
    xd = gxd[...]
    xs = gxs[...]
    ev = e[...]
    cat = jnp.concatenate([xd, xs, ev], axis=1)
    h = jnp.maximum(_dot(cat, w1[...]) + b1[...], 0.0)
    h = jnp.maximum(_dot(h, w2[...]) + b2[...], 0.0)
    h = _dot(h, w3[...]) + b3[...]
    e_new = _ln(h, g[...], bt[...])
    e1_out[...] = ev + e_new
    qk = _dot(jnp.concatenate([xd, xs], axis=1), wqk[...])
    prod = qk[:, :H] * qk[:, H:]
    p = jnp.exp(_dot(prod, sel[...]))
    v = _dot(e_new, wv[...])
    u_out[...] = _dot(p, exp16[...]) * v
    p_out[...] = jnp.concatenate(
        [p, jnp.zeros((p.shape[0], H - 16), _F32)], axis=1)


def _edge_block(gxd, gxs, e, w1, b1, w2, b2, w3, b3, g, bt, wqk, wv, sel, exp16):
    grid = E // TT
    wspec = lambda a: pl.BlockSpec(a.shape, lambda i: (0,) * a.ndim)
    ws = (w1, b1, w2, b2, w3, b3, g, bt, wqk, wv, sel, exp16)
    return pl.pallas_call(
        _edge_body,
        grid=(grid,),
        in_specs=[pl.BlockSpec((TT, H), lambda i: (i, 0))] * 3
        + [wspec(a) for a in ws],
        out_specs=[
            pl.BlockSpec((TT, H), lambda i: (i, 0)),
            pl.BlockSpec((TT, H), lambda i: (i, 0)),
            pl.BlockSpec((TT, H), lambda i: (i, 0)),
        ],
        out_shape=[
            jax.ShapeDtypeStruct((E, H), _F32),
            jax.ShapeDtypeStruct((E, H), _F32),
            jax.ShapeDtypeStruct((E, H), _F32),
        ],
    )(gxd, gxs, e, *ws)


def _node_body(dec, x, accu, accp, w1, b1, w2, b2, w3, b3, g, bt, wo, expb,
               *rest):
    if dec:
        (wd1, bd1, wd2, bd2, wd3, bd3, out) = rest
    else:
        (out,) = rest
    xv = x[...]
    u = accu[...]
    pm = accp[...]
    rec = 1.0 / (pm + 1e-12)
    recb = _dot(rec, expb[...])
    msg = _dot(u * recb, wo[...])
    cat = jnp.concatenate([xv, msg], axis=1)
    h = jnp.maximum(_dot(cat, w1[...]) + b1[...], 0.0)
    h = jnp.maximum(_dot(h, w2[...]) + b2[...], 0.0)
    h = _dot(h, w3[...]) + b3[...]
    xn = xv + _ln(h, g[...], bt[...])
    if dec:
        h = jnp.maximum(_dot(xn, wd1[...]) + bd1[...], 0.0)
        h = jnp.maximum(_dot(h, wd2[...]) + bd2[...], 0.0)
        out[...] = _dot(h, wd3[...]) + bd3[...]
    else:
        out[...] = xn


def _node_block(x, accu, accp, weights, dec_weights=None):
    grid = N // TT
    wspec = lambda a: pl.BlockSpec(a.shape, lambda i: (0,) * a.ndim)
    dec = dec_weights is not None
    ws = tuple(weights) + (tuple(dec_weights) if dec else ())
    return pl.pallas_call(
        functools.partial(_node_body, dec),
        grid=(grid,),
        in_specs=[
            pl.BlockSpec((TT, H), lambda i: (i, 0)),
            pl.BlockSpec((TT, H), lambda i: (i, 0)),
            pl.BlockSpec((TT, H), lambda i: (i, 0)),
        ]
        + [wspec(a) for a in ws],
        out_specs=pl.BlockSpec((TT, H), lambda i: (i, 0)),
        out_shape=jax.ShapeDtypeStruct((N, H), _F32),
    )(x, accu, accp, *ws)


def _sc_gather2(table, idx_a, idx_b):
    d = table.shape[1]
    ci = 1000
    cg = 200
    per_w = E // NW
    nch = per_w // ci
    mesh = plsc.VectorSubcoreMesh(core_axis_name="c", subcore_axis_name="s")

    @functools.partial(
        pl.kernel,
        mesh=mesh,
        out_type=[jax.ShapeDtypeStruct((E, d), _F32)] * 2,
        scratch_types=[
            pltpu.VMEM((ci,), jnp.int32),
            pltpu.VMEM((cg, d), _F32),
            pltpu.SemaphoreType.DMA,
        ],
    )
    def k(tab_hbm, ia_hbm, ib_hbm, oa_hbm, ob_hbm, idx_v, rows_v, sem):
        wid = jax.lax.axis_index("s") * NC + jax.lax.axis_index("c")

        @pl.loop(0, nch)
        def _(j):
            base = wid * per_w + j * ci

            def one(i_hbm, o_hbm):
                pltpu.sync_copy(i_hbm.at[pl.ds(base, ci)], idx_v)
                for h in range(ci // cg):
                    pltpu.async_copy(
                        tab_hbm.at[idx_v.at[pl.ds(h * cg, cg)]], rows_v, sem
                    ).wait()
                    pltpu.sync_copy(rows_v, o_hbm.at[pl.ds(base + h * cg, cg)])

            one(ia_hbm, oa_hbm)
            one(ib_hbm, ob_hbm)

    return k(table, idx_a, idx_b)


def _sc_scatter_add(u, p128, dst, zu):
    c = 200
    cs = 40
    nsb = c // cs
    per_s = E // NS
    nch = per_s // c
    rps = NPAD // NS
    mesh = plsc.VectorSubcoreMesh(core_axis_name="c", subcore_axis_name="s")

    @functools.partial(
        pl.kernel,
        mesh=mesh,
        out_type=[
            jax.ShapeDtypeStruct((NPAD, H), _F32),
            jax.ShapeDtypeStruct((NPAD, H), _F32),
        ],
        scratch_types=[
            pltpu.VMEM((c,), jnp.int32),
            pltpu.VMEM((nsb, cs), jnp.int32),
            pltpu.VMEM((c, H), _F32),
            pltpu.VMEM_SHARED((NPAD, H), _F32),
            pltpu.SemaphoreType.DMA,
        ],
    )
    def k(u_hbm, p_hbm, dst_hbm, zu_hbm, ou_hbm, op_hbm,
          idx_v, idx2_v, d_v, acc, sem):
        ci = jax.lax.axis_index("c")
        si = jax.lax.axis_index("s")
        row0 = si * rps
        pltpu.sync_copy(zu_hbm.at[pl.ds(row0, rps)], acc.at[pl.ds(row0, rps)])
        plsc.subcore_barrier()

        def body(src_hbm):
            @pl.loop(0, nch)
            def _(j):
                base = si * per_s + j * c
                pltpu.sync_copy(dst_hbm.at[pl.ds(base, c)], idx_v)
                pltpu.sync_copy(src_hbm.at[pl.ds(base, c)], d_v)

                for sb in range(nsb):
                    for kk in list(range(0, cs - 15, 16)) + (
                            [cs - 16] if cs % 16 else []):
                        idx2_v[sb, pl.ds(kk, 16)] = (
                            idx_v[pl.ds(sb * cs + kk, 16)])

                cps = [pltpu.async_copy(
                    d_v.at[pl.ds(sb * cs, cs)],
                    acc.at[idx2_v.at[sb]], sem, add=True)
                    for sb in range(nsb)]
                for cp in cps:
                    cp.wait()

        @pl.when(ci == 0)
        def _():
            body(u_hbm)

        @pl.when(ci == 1)
        def _():
            body(p_hbm)

        plsc.subcore_barrier()

        @pl.when(ci == 0)
        def _():
            pltpu.sync_copy(acc.at[pl.ds(row0, rps)],
                            ou_hbm.at[pl.ds(row0, rps)])

        @pl.when(ci == 1)
        def _():
            pltpu.sync_copy(acc.at[pl.ds(row0, rps)],
                            op_hbm.at[pl.ds(row0, rps)])

    return k(u, p128, dst, zu)


def _r1(v):
    return v.reshape(1, -1)


def kernel(field, pos, bc_type, edge_index, face_normals, face_areas,
           face_type, params):
    src = edge_index[0].astype(jnp.int32)
    dst = edge_index[1].astype(jnp.int32)

    lane = jax.lax.broadcasted_iota(jnp.int32, (H, 16), 0) // 16
    head = jax.lax.broadcasted_iota(jnp.int32, (H, 16), 1)
    sel = jnp.where(lane == head, 0.25, 0.0).astype(_F32)
    exp16 = sel.T * 4.0

    ne = params['node_enc']
    dist_lo = jnp.clip(pos, -1.0, 1.0)
    dist_hi = jnp.clip(1.0 - pos, -1.0, 1.0)
    bcoh = (bc_type[:, None] == jnp.arange(5)[None, :]).astype(_F32)
    nf16 = jnp.concatenate(
        [field, dist_lo, dist_hi, bcoh, jnp.zeros((N, 3), _F32)], axis=1)
    wn1 = ne['linears'][0]['W']
    wn16 = jnp.concatenate(
        [wn1[:8], params['bc_embed'] @ wn1[8:16], jnp.zeros((3, H), _F32)], axis=0)
    x = _mlp16(nf16, wn16, _r1(ne['linears'][0]['b']),
               ne['linears'][1]['W'], _r1(ne['linears'][1]['b']),
               ne['linears'][2]['W'], _r1(ne['linears'][2]['b']),
               _r1(ne['ln']['g']), _r1(ne['ln']['b']))

    pos128 = jnp.concatenate([pos, jnp.zeros((N, H - 2), _F32)], axis=1)
    pd, ps = _sc_gather2(pos128, dst, src)
    delta = pd[:, :2] - ps[:, :2]
    dist = jnp.maximum(
        jnp.sqrt(jnp.sum(delta * delta, axis=1, keepdims=True)), 1e-8)
    unit = delta / dist
    ftoh = (face_type[:, None] == jnp.arange(4)[None, :]).astype(_F32)
    ef16 = jnp.concatenate(
        [face_normals, face_areas[:, None], dist, unit, ftoh,
         jnp.zeros((E, 6), _F32)], axis=1)
    ee = params['edge_enc']
    we1 = ee['linears'][0]['W']
    we16 = jnp.concatenate(
        [we1[:6], params['ft_embed'] @ we1[6:10], jnp.zeros((6, H), _F32)], axis=0)
    e = _mlp16(ef16, we16, _r1(ee['linears'][0]['b']),
               ee['linears'][1]['W'], _r1(ee['linears'][1]['b']),
               ee['linears'][2]['W'], _r1(ee['linears'][2]['b']),
               _r1(ee['ln']['g']), _r1(ee['ln']['b']))

    zu = jnp.zeros((NPAD, H), _F32)
    expb128 = jnp.concatenate([exp16, jnp.zeros((H - 16, H), _F32)], axis=0)
    dec = params['dec']
    dec_ws = (dec['linears'][0]['W'], _r1(dec['linears'][0]['b']),
              dec['linears'][1]['W'], _r1(dec['linears'][1]['b']),
              jnp.pad(dec['linears'][2]['W'], ((0, 0), (0, H - 4))),
              _r1(jnp.pad(dec['linears'][2]['b'], (0, H - 4))))

    for b, bp in enumerate(params['blocks']):
        gxd, gxs = _sc_gather2(x, dst, src)
        em = bp['edge_mlp']
        wqk = jnp.block([[bp['W_Q'], jnp.zeros((H, H), _F32)],
                         [jnp.zeros((H, H), _F32), bp['W_K']]])
        e_next, u, p128 = _edge_block(
            gxd, gxs, e,
            em['linears'][0]['W'], _r1(em['linears'][0]['b']),
            em['linears'][1]['W'], _r1(em['linears'][1]['b']),
            em['linears'][2]['W'], _r1(em['linears'][2]['b']),
            _r1(em['ln']['g']), _r1(em['ln']['b']),
            wqk, bp['W_V'], sel, exp16)
        accu, accp = _sc_scatter_add(u, p128, dst, zu)
        nm = bp['node_mlp']
        node_ws = (nm['linears'][0]['W'], _r1(nm['linears'][0]['b']),
                   nm['linears'][1]['W'], _r1(nm['linears'][1]['b']),
                   nm['linears'][2]['W'], _r1(nm['linears'][2]['b']),
                   _r1(nm['ln']['g']), _r1(nm['ln']['b']),
                   bp['W_O'], expb128)
        x = _node_block(x, accu, accp, node_ws,
                        dec_weights=dec_ws if b == 1 else None)
        e = e_next

    return x[:, :4]


# speedup vs baseline: 5.3904x; 1.2242x over previous
import dataclasses
import functools

import jax
import jax.numpy as jnp
from jax.experimental import pallas as pl
from jax.experimental.pallas import tpu as pltpu
from jax.experimental.pallas import tpu_sc as plsc

N = 10000
E = 160000
H = 128
NHEADS = 8
NPAD = 10240
NC, NS = 2, 16
NW = NC * NS
TT = 2000

_F32 = jnp.float32


def _ln(h, g, b):
    mu = jnp.mean(h, axis=1, keepdims=True)
    var = jnp.mean((h - mu) ** 2, axis=1, keepdims=True)
    return (h - mu) * jax.lax.rsqrt(var + 1e-5) * g + b


def _dot(a, b):
    return jnp.dot(a, b, preferred_element_type=_F32)


def _enc_body(xin, w1, b1, w2, b2, w3, b3, g, bt, out):
    h = jnp.maximum(_dot(xin[...], w1[...]) + b1[...], 0.0)
    h = jnp.maximum(_dot(h, w2[...]) + b2[...], 0.0)
    h = _dot(h, w3[...]) + b3[...]
    out[...] = _ln(h, g[...], bt[...])


def _mlp16(xin, w1, b1, w2, b2, w3, b3, g, bt):
    rows = xin.shape[0]
    grid = rows // TT
    wspec = lambda a: pl.BlockSpec(a.shape, lambda i: (0,) * a.ndim)
    return pl.pallas_call(
        _enc_body,
        grid=(grid,),
        in_specs=[pl.BlockSpec((TT, 16), lambda i: (i, 0))]
        + [wspec(a) for a in (w1, b1, w2, b2, w3, b3, g, bt)],
        out_specs=pl.BlockSpec((TT, H), lambda i: (i, 0)),
        out_shape=jax.ShapeDtypeStruct((rows, H), _F32),
    )(xin, w1, b1, w2, b2, w3, b3, g, bt)


def _edge_body(gxd, gxs, e, w1, b1, w2, b2, w3, b3, g, bt, wqk, wv, sel, exp16,
               e1_out, u_out, p_out):
    xd = gxd[...]
    xs = gxs[...]
    ev = e[...]
    cat = jnp.concatenate([xd, xs, ev], axis=1)
    h = jnp.maximum(_dot(cat, w1[...]) + b1[...], 0.0)
    h = jnp.maximum(_dot(h, w2[...]) + b2[...], 0.0)
    h = _dot(h, w3[...]) + b3[...]
    e_new = _ln(h, g[...], bt[...])
    e1_out[...] = ev + e_new
    qk = _dot(jnp.concatenate([xd, xs], axis=1), wqk[...])
    prod = qk[:, :H] * qk[:, H:]
    p = jnp.exp(_dot(prod, sel[...]))
    v = _dot(e_new, wv[...])
    u_out[...] = _dot(p, exp16[...]) * v
    p_out[...] = jnp.concatenate(
        [p, jnp.zeros((p.shape[0], H - 16), _F32)], axis=1)


def _edge_block(gxd, gxs, e, w1, b1, w2, b2, w3, b3, g, bt, wqk, wv, sel, exp16):
    grid = E // TT
    wspec = lambda a: pl.BlockSpec(a.shape, lambda i: (0,) * a.ndim)
    ws = (w1, b1, w2, b2, w3, b3, g, bt, wqk, wv, sel, exp16)
    return pl.pallas_call(
        _edge_body,
        grid=(grid,),
        in_specs=[pl.BlockSpec((TT, H), lambda i: (i, 0))] * 3
        + [wspec(a) for a in ws],
        out_specs=[
            pl.BlockSpec((TT, H), lambda i: (i, 0)),
            pl.BlockSpec((TT, H), lambda i: (i, 0)),
            pl.BlockSpec((TT, H), lambda i: (i, 0)),
        ],
        out_shape=[
            jax.ShapeDtypeStruct((E, H), _F32),
            jax.ShapeDtypeStruct((E, H), _F32),
            jax.ShapeDtypeStruct((E, H), _F32),
        ],
    )(gxd, gxs, e, *ws)


def _node_body(dec, x, accu, accp, w1, b1, w2, b2, w3, b3, g, bt, wo, expb,
               *rest):
    if dec:
        (wd1, bd1, wd2, bd2, wd3, bd3, out) = rest
    else:
        (out,) = rest
    xv = x[...]
    u = accu[...]
    pm = accp[...]
    rec = 1.0 / (pm + 1e-12)
    recb = _dot(rec, expb[...])
    msg = _dot(u * recb, wo[...])
    cat = jnp.concatenate([xv, msg], axis=1)
    h = jnp.maximum(_dot(cat, w1[...]) + b1[...], 0.0)
    h = jnp.maximum(_dot(h, w2[...]) + b2[...], 0.0)
    h = _dot(h, w3[...]) + b3[...]
    xn = xv + _ln(h, g[...], bt[...])
    if dec:
        h = jnp.maximum(_dot(xn, wd1[...]) + bd1[...], 0.0)
        h = jnp.maximum(_dot(h, wd2[...]) + bd2[...], 0.0)
        out[...] = _dot(h, wd3[...]) + bd3[...]
    else:
        out[...] = xn


def _node_block(x, accu, accp, weights, dec_weights=None):
    grid = N // TT
    wspec = lambda a: pl.BlockSpec(a.shape, lambda i: (0,) * a.ndim)
    dec = dec_weights is not None
    ws = tuple(weights) + (tuple(dec_weights) if dec else ())
    return pl.pallas_call(
        functools.partial(_node_body, dec),
        grid=(grid,),
        in_specs=[
            pl.BlockSpec((TT, H), lambda i: (i, 0)),
            pl.BlockSpec((TT, H), lambda i: (i, 0)),
            pl.BlockSpec((TT, H), lambda i: (i, 0)),
        ]
        + [wspec(a) for a in ws],
        out_specs=pl.BlockSpec((TT, H), lambda i: (i, 0)),
        out_shape=jax.ShapeDtypeStruct((N, H), _F32),
    )(x, accu, accp, *ws)


def _sc_gather2(table, idx_a, idx_b):
    d = table.shape[1]
    ci = 1000
    cg = 200
    per_w = E // NW
    nch = per_w // ci
    mesh = plsc.VectorSubcoreMesh(core_axis_name="c", subcore_axis_name="s")

    @functools.partial(
        pl.kernel,
        mesh=mesh,
        out_type=[jax.ShapeDtypeStruct((E, d), _F32)] * 2,
        scratch_types=[
            pltpu.VMEM((ci,), jnp.int32),
            pltpu.VMEM((cg, d), _F32),
            pltpu.SemaphoreType.DMA,
        ],
    )
    def k(tab_hbm, ia_hbm, ib_hbm, oa_hbm, ob_hbm, idx_v, rows_v, sem):
        wid = jax.lax.axis_index("s") * NC + jax.lax.axis_index("c")

        @pl.loop(0, nch)
        def _(j):
            base = wid * per_w + j * ci

            def one(i_hbm, o_hbm):
                pltpu.sync_copy(i_hbm.at[pl.ds(base, ci)], idx_v)
                for h in range(ci // cg):
                    pltpu.async_copy(
                        tab_hbm.at[idx_v.at[pl.ds(h * cg, cg)]], rows_v, sem
                    ).wait()
                    pltpu.sync_copy(rows_v, o_hbm.at[pl.ds(base + h * cg, cg)])

            one(ia_hbm, oa_hbm)
            one(ib_hbm, ob_hbm)

    return k(table, idx_a, idx_b)


def _sc_pos_delta(px, py, dst, src):
    per_w = E // NW
    mesh = plsc.VectorSubcoreMesh(core_axis_name="c", subcore_axis_name="s")
    cp = pltpu.CompilerParams()
    if "needs_layout_passes" in pltpu.CompilerParams.__dataclass_fields__:
        cp = dataclasses.replace(cp, needs_layout_passes=False)

    @functools.partial(
        pl.kernel,
        mesh=mesh,
        compiler_params=cp,
        out_type=[jax.ShapeDtypeStruct((E,), _F32)] * 2,
        scratch_types=[
            pltpu.VMEM((N,), _F32),
            pltpu.VMEM((N,), _F32),
            pltpu.VMEM((per_w,), jnp.int32),
            pltpu.VMEM((per_w,), jnp.int32),
            pltpu.VMEM((per_w,), _F32),
            pltpu.VMEM((per_w,), _F32),
        ],
    )
    def k(px_hbm, py_hbm, dst_hbm, src_hbm, dx_hbm, dy_hbm,
          px_v, py_v, id_v, is_v, dx_v, dy_v):
        wid = jax.lax.axis_index("s") * NC + jax.lax.axis_index("c")
        base = wid * per_w
        pltpu.sync_copy(px_hbm, px_v)
        pltpu.sync_copy(py_hbm, py_v)
        pltpu.sync_copy(dst_hbm.at[pl.ds(base, per_w)], id_v)
        pltpu.sync_copy(src_hbm.at[pl.ds(base, per_w)], is_v)

        def body(kk):
            d = id_v[pl.ds(kk, 16)]
            s = is_v[pl.ds(kk, 16)]
            dx_v[pl.ds(kk, 16)] = (plsc.load_gather(px_v, [d])
                                   - plsc.load_gather(px_v, [s]))
            dy_v[pl.ds(kk, 16)] = (plsc.load_gather(py_v, [d])
                                   - plsc.load_gather(py_v, [s]))

        nfull = per_w // 16 * 16

        @pl.loop(0, nfull, step=16)
        def _(kk):
            body(kk)

        if per_w % 16:
            body(per_w - 16)

        pltpu.sync_copy(dx_v, dx_hbm.at[pl.ds(base, per_w)])
        pltpu.sync_copy(dy_v, dy_hbm.at[pl.ds(base, per_w)])

    return k(px, py, dst, src)


def _sc_scatter_add(u, p128, dst, zu):
    c = 200
    cs = 40
    nsb = c // cs
    per_s = E // NS
    nch = per_s // c
    rps = NPAD // NS
    mesh = plsc.VectorSubcoreMesh(core_axis_name="c", subcore_axis_name="s")

    @functools.partial(
        pl.kernel,
        mesh=mesh,
        out_type=[
            jax.ShapeDtypeStruct((NPAD, H), _F32),
            jax.ShapeDtypeStruct((NPAD, H), _F32),
        ],
        scratch_types=[
            pltpu.VMEM((c,), jnp.int32),
            pltpu.VMEM((nsb, cs), jnp.int32),
            pltpu.VMEM((c, H), _F32),
            pltpu.VMEM_SHARED((NPAD, H), _F32),
            pltpu.SemaphoreType.DMA,
        ],
    )
    def k(u_hbm, p_hbm, dst_hbm, zu_hbm, ou_hbm, op_hbm,
          idx_v, idx2_v, d_v, acc, sem):
        ci = jax.lax.axis_index("c")
        si = jax.lax.axis_index("s")
        row0 = si * rps
        pltpu.sync_copy(zu_hbm.at[pl.ds(row0, rps)], acc.at[pl.ds(row0, rps)])
        plsc.subcore_barrier()

        def body(src_hbm):
            @pl.loop(0, nch)
            def _(j):
                base = si * per_s + j * c
                pltpu.sync_copy(dst_hbm.at[pl.ds(base, c)], idx_v)
                pltpu.sync_copy(src_hbm.at[pl.ds(base, c)], d_v)

                for sb in range(nsb):
                    for kk in list(range(0, cs - 15, 16)) + (
                            [cs - 16] if cs % 16 else []):
                        idx2_v[sb, pl.ds(kk, 16)] = (
                            idx_v[pl.ds(sb * cs + kk, 16)])

                cps = [pltpu.async_copy(
                    d_v.at[pl.ds(sb * cs, cs)],
                    acc.at[idx2_v.at[sb]], sem, add=True)
                    for sb in range(nsb)]
                for cp in cps:
                    cp.wait()

        @pl.when(ci == 0)
        def _():
            body(u_hbm)

        @pl.when(ci == 1)
        def _():
            body(p_hbm)

        plsc.subcore_barrier()

        @pl.when(ci == 0)
        def _():
            pltpu.sync_copy(acc.at[pl.ds(row0, rps)],
                            ou_hbm.at[pl.ds(row0, rps)])

        @pl.when(ci == 1)
        def _():
            pltpu.sync_copy(acc.at[pl.ds(row0, rps)],
                            op_hbm.at[pl.ds(row0, rps)])

    return k(u, p128, dst, zu)


def _r1(v):
    return v.reshape(1, -1)


def kernel(field, pos, bc_type, edge_index, face_normals, face_areas,
           face_type, params):
    src = edge_index[0].astype(jnp.int32)
    dst = edge_index[1].astype(jnp.int32)

    lane = jax.lax.broadcasted_iota(jnp.int32, (H, 16), 0) // 16
    head = jax.lax.broadcasted_iota(jnp.int32, (H, 16), 1)
    sel = jnp.where(lane == head, 0.25, 0.0).astype(_F32)
    exp16 = sel.T * 4.0

    ne = params['node_enc']
    dist_lo = jnp.clip(pos, -1.0, 1.0)
    dist_hi = jnp.clip(1.0 - pos, -1.0, 1.0)
    bcoh = (bc_type[:, None] == jnp.arange(5)[None, :]).astype(_F32)
    nf16 = jnp.concatenate(
        [field, dist_lo, dist_hi, bcoh, jnp.zeros((N, 3), _F32)], axis=1)
    wn1 = ne['linears'][0]['W']
    wn16 = jnp.concatenate(
        [wn1[:8], params['bc_embed'] @ wn1[8:16], jnp.zeros((3, H), _F32)], axis=0)
    x = _mlp16(nf16, wn16, _r1(ne['linears'][0]['b']),
               ne['linears'][1]['W'], _r1(ne['linears'][1]['b']),
               ne['linears'][2]['W'], _r1(ne['linears'][2]['b']),
               _r1(ne['ln']['g']), _r1(ne['ln']['b']))

    dx, dy = _sc_pos_delta(pos[:, 0], pos[:, 1], dst, src)
    delta = jnp.stack([dx, dy], axis=1)
    dist = jnp.maximum(
        jnp.sqrt(jnp.sum(delta * delta, axis=1, keepdims=True)), 1e-8)
    unit = delta / dist
    ftoh = (face_type[:, None] == jnp.arange(4)[None, :]).astype(_F32)
    ef16 = jnp.concatenate(
        [face_normals, face_areas[:, None], dist, unit, ftoh,
         jnp.zeros((E, 6), _F32)], axis=1)
    ee = params['edge_enc']
    we1 = ee['linears'][0]['W']
    we16 = jnp.concatenate(
        [we1[:6], params['ft_embed'] @ we1[6:10], jnp.zeros((6, H), _F32)], axis=0)
    e = _mlp16(ef16, we16, _r1(ee['linears'][0]['b']),
               ee['linears'][1]['W'], _r1(ee['linears'][1]['b']),
               ee['linears'][2]['W'], _r1(ee['linears'][2]['b']),
               _r1(ee['ln']['g']), _r1(ee['ln']['b']))

    zu = jnp.zeros((NPAD, H), _F32)
    expb128 = jnp.concatenate([exp16, jnp.zeros((H - 16, H), _F32)], axis=0)
    dec = params['dec']
    dec_ws = (dec['linears'][0]['W'], _r1(dec['linears'][0]['b']),
              dec['linears'][1]['W'], _r1(dec['linears'][1]['b']),
              jnp.pad(dec['linears'][2]['W'], ((0, 0), (0, H - 4))),
              _r1(jnp.pad(dec['linears'][2]['b'], (0, H - 4))))

    for b, bp in enumerate(params['blocks']):
        gxd, gxs = _sc_gather2(x, dst, src)
        em = bp['edge_mlp']
        wqk = jnp.block([[bp['W_Q'], jnp.zeros((H, H), _F32)],
                         [jnp.zeros((H, H), _F32), bp['W_K']]])
        e_next, u, p128 = _edge_block(
            gxd, gxs, e,
            em['linears'][0]['W'], _r1(em['linears'][0]['b']),
            em['linears'][1]['W'], _r1(em['linears'][1]['b']),
            em['linears'][2]['W'], _r1(em['linears'][2]['b']),
            _r1(em['ln']['g']), _r1(em['ln']['b']),
            wqk, bp['W_V'], sel, exp16)
        accu, accp = _sc_scatter_add(u, p128, dst, zu)
        nm = bp['node_mlp']
        node_ws = (nm['linears'][0]['W'], _r1(nm['linears'][0]['b']),
                   nm['linears'][1]['W'], _r1(nm['linears'][1]['b']),
                   nm['linears'][2]['W'], _r1(nm['linears'][2]['b']),
                   _r1(nm['ln']['g']), _r1(nm['ln']['b']),
                   bp['W_O'], expb128)
        x = _node_block(x, accu, accp, node_ws,
                        dec_weights=dec_ws if b == 1 else None)
        e = e_next

    return x[:, :4]


# speedup vs baseline: 5.4920x; 1.0188x over previous
import dataclasses
import functools

import jax
import jax.numpy as jnp
from jax.experimental import pallas as pl
from jax.experimental.pallas import tpu as pltpu
from jax.experimental.pallas import tpu_sc as plsc

N = 10000
E = 160000
H = 128
NHEADS = 8
NPAD = 10240
NC, NS = 2, 16
NW = NC * NS
TT = 2000

_F32 = jnp.float32


def _ln(h, g, b):
    mu = jnp.mean(h, axis=1, keepdims=True)
    var = jnp.mean((h - mu) ** 2, axis=1, keepdims=True)
    return (h - mu) * jax.lax.rsqrt(var + 1e-5) * g + b


def _dot(a, b):
    return jnp.dot(a, b, preferred_element_type=_F32)


def _enc_body(xin, w1, b1, w2, b2, w3, b3, g, bt, out):
    h = jnp.maximum(_dot(xin[...], w1[...]) + b1[...], 0.0)
    h = jnp.maximum(_dot(h, w2[...]) + b2[...], 0.0)
    h = _dot(h, w3[...]) + b3[...]
    out[...] = _ln(h, g[...], bt[...])


def _mlp16(xin, w1, b1, w2, b2, w3, b3, g, bt):
    rows = xin.shape[0]
    grid = rows // TT
    wspec = lambda a: pl.BlockSpec(a.shape, lambda i: (0,) * a.ndim)
    return pl.pallas_call(
        _enc_body,
        grid=(grid,),
        in_specs=[pl.BlockSpec((TT, 16), lambda i: (i, 0))]
        + [wspec(a) for a in (w1, b1, w2, b2, w3, b3, g, bt)],
        out_specs=pl.BlockSpec((TT, H), lambda i: (i, 0)),
        out_shape=jax.ShapeDtypeStruct((rows, H), _F32),
    )(xin, w1, b1, w2, b2, w3, b3, g, bt)


def _edge_body(emit_e, gxd, gxs, e, w1, b1, w2, b2, w3, b3, g, bt, wqk, wv,
               sel, exp16, *outs):
    if emit_e:
        e1_out, u_out, p_out = outs
    else:
        u_out, p_out = outs
    xd = gxd[...]
    xs = gxs[...]
    ev = e[...]
    cat = jnp.concatenate([xd, xs, ev], axis=1)
    h = jnp.maximum(_dot(cat, w1[...]) + b1[...], 0.0)
    h = jnp.maximum(_dot(h, w2[...]) + b2[...], 0.0)
    h = _dot(h, w3[...]) + b3[...]
    e_new = _ln(h, g[...], bt[...])
    if emit_e:
        e1_out[...] = ev + e_new
    qk = _dot(jnp.concatenate([xd, xs], axis=1), wqk[...])
    prod = qk[:, :H] * qk[:, H:]
    p = jnp.exp(_dot(prod, sel[...]))
    v = _dot(e_new, wv[...])
    u_out[...] = _dot(p, exp16[...]) * v
    p_out[...] = jnp.concatenate(
        [p, jnp.zeros((p.shape[0], H - 16), _F32)], axis=1)


def _edge_block(emit_e, gxd, gxs, e, w1, b1, w2, b2, w3, b3, g, bt, wqk, wv,
                sel, exp16):
    ee = gxd.shape[0]
    grid = ee // TT
    wspec = lambda a: pl.BlockSpec(a.shape, lambda i: (0,) * a.ndim)
    ws = (w1, b1, w2, b2, w3, b3, g, bt, wqk, wv, sel, exp16)
    nout = 3 if emit_e else 2
    return pl.pallas_call(
        functools.partial(_edge_body, emit_e),
        grid=(grid,),
        in_specs=[pl.BlockSpec((TT, H), lambda i: (i, 0))] * 3
        + [wspec(a) for a in ws],
        out_specs=[pl.BlockSpec((TT, H), lambda i: (i, 0))] * nout,
        out_shape=[jax.ShapeDtypeStruct((ee, H), _F32)] * nout,
    )(gxd, gxs, e, *ws)


def _node_body(dec, x, accu, accp, w1, b1, w2, b2, w3, b3, g, bt, wo, expb,
               *rest):
    if dec:
        (wd1, bd1, wd2, bd2, wd3, bd3, out) = rest
    else:
        (out,) = rest
    xv = x[...]
    u = accu[...]
    pm = accp[...]
    rec = 1.0 / (pm + 1e-12)
    recb = _dot(rec, expb[...])
    msg = _dot(u * recb, wo[...])
    cat = jnp.concatenate([xv, msg], axis=1)
    h = jnp.maximum(_dot(cat, w1[...]) + b1[...], 0.0)
    h = jnp.maximum(_dot(h, w2[...]) + b2[...], 0.0)
    h = _dot(h, w3[...]) + b3[...]
    xn = xv + _ln(h, g[...], bt[...])
    if dec:
        h = jnp.maximum(_dot(xn, wd1[...]) + bd1[...], 0.0)
        h = jnp.maximum(_dot(h, wd2[...]) + bd2[...], 0.0)
        out[...] = _dot(h, wd3[...]) + bd3[...]
    else:
        out[...] = xn


def _node_block(x, accu, accp, weights, dec_weights=None):
    grid = N // TT
    wspec = lambda a: pl.BlockSpec(a.shape, lambda i: (0,) * a.ndim)
    dec = dec_weights is not None
    ws = tuple(weights) + (tuple(dec_weights) if dec else ())
    return pl.pallas_call(
        functools.partial(_node_body, dec),
        grid=(grid,),
        in_specs=[
            pl.BlockSpec((TT, H), lambda i: (i, 0)),
            pl.BlockSpec((TT, H), lambda i: (i, 0)),
            pl.BlockSpec((TT, H), lambda i: (i, 0)),
        ]
        + [wspec(a) for a in ws],
        out_specs=pl.BlockSpec((TT, H), lambda i: (i, 0)),
        out_shape=jax.ShapeDtypeStruct((N, H), _F32),
    )(x, accu, accp, *ws)


def _sc_gather2(table, idx_a, idx_b):
    d = table.shape[1]
    ci = 1000
    cg = 200
    per_w = E // NW
    nch = per_w // ci
    mesh = plsc.VectorSubcoreMesh(core_axis_name="c", subcore_axis_name="s")

    @functools.partial(
        pl.kernel,
        mesh=mesh,
        out_type=[jax.ShapeDtypeStruct((E, d), _F32)] * 2,
        scratch_types=[
            pltpu.VMEM((ci,), jnp.int32),
            pltpu.VMEM((cg, d), _F32),
            pltpu.SemaphoreType.DMA,
        ],
    )
    def k(tab_hbm, ia_hbm, ib_hbm, oa_hbm, ob_hbm, idx_v, rows_v, sem):
        wid = jax.lax.axis_index("s") * NC + jax.lax.axis_index("c")

        @pl.loop(0, nch)
        def _(j):
            base = wid * per_w + j * ci

            def one(i_hbm, o_hbm):
                pltpu.sync_copy(i_hbm.at[pl.ds(base, ci)], idx_v)
                for h in range(ci // cg):
                    pltpu.async_copy(
                        tab_hbm.at[idx_v.at[pl.ds(h * cg, cg)]], rows_v, sem
                    ).wait()
                    pltpu.sync_copy(rows_v, o_hbm.at[pl.ds(base + h * cg, cg)])

            one(ia_hbm, oa_hbm)
            one(ib_hbm, ob_hbm)

    return k(table, idx_a, idx_b)


def _sc_pos_delta(px, py, dst, src):
    per_w = E // NW
    mesh = plsc.VectorSubcoreMesh(core_axis_name="c", subcore_axis_name="s")
    cp = pltpu.CompilerParams()
    if "needs_layout_passes" in pltpu.CompilerParams.__dataclass_fields__:
        cp = dataclasses.replace(cp, needs_layout_passes=False)

    @functools.partial(
        pl.kernel,
        mesh=mesh,
        compiler_params=cp,
        out_type=[jax.ShapeDtypeStruct((E,), _F32)] * 2,
        scratch_types=[
            pltpu.VMEM((N,), _F32),
            pltpu.VMEM((N,), _F32),
            pltpu.VMEM((per_w,), jnp.int32),
            pltpu.VMEM((per_w,), jnp.int32),
            pltpu.VMEM((per_w,), _F32),
            pltpu.VMEM((per_w,), _F32),
        ],
    )
    def k(px_hbm, py_hbm, dst_hbm, src_hbm, dx_hbm, dy_hbm,
          px_v, py_v, id_v, is_v, dx_v, dy_v):
        wid = jax.lax.axis_index("s") * NC + jax.lax.axis_index("c")
        base = wid * per_w
        pltpu.sync_copy(px_hbm, px_v)
        pltpu.sync_copy(py_hbm, py_v)
        pltpu.sync_copy(dst_hbm.at[pl.ds(base, per_w)], id_v)
        pltpu.sync_copy(src_hbm.at[pl.ds(base, per_w)], is_v)

        def body(kk):
            d = id_v[pl.ds(kk, 16)]
            s = is_v[pl.ds(kk, 16)]
            dx_v[pl.ds(kk, 16)] = (plsc.load_gather(px_v, [d])
                                   - plsc.load_gather(px_v, [s]))
            dy_v[pl.ds(kk, 16)] = (plsc.load_gather(py_v, [d])
                                   - plsc.load_gather(py_v, [s]))

        nfull = per_w // 16 * 16

        @pl.loop(0, nfull, step=16)
        def _(kk):
            body(kk)

        if per_w % 16:
            body(per_w - 16)

        pltpu.sync_copy(dx_v, dx_hbm.at[pl.ds(base, per_w)])
        pltpu.sync_copy(dy_v, dy_hbm.at[pl.ds(base, per_w)])

    return k(px, py, dst, src)


def _sc_scatter_add(u, p128, dst, zu):
    c = 200
    cs = 40
    nsb = c // cs
    per_s = E // NS
    nch = per_s // c
    rps = NPAD // NS
    mesh = plsc.VectorSubcoreMesh(core_axis_name="c", subcore_axis_name="s")

    @functools.partial(
        pl.kernel,
        mesh=mesh,
        out_type=[
            jax.ShapeDtypeStruct((NPAD, H), _F32),
            jax.ShapeDtypeStruct((NPAD, H), _F32),
        ],
        scratch_types=[
            pltpu.VMEM((c,), jnp.int32),
            pltpu.VMEM((nsb, cs), jnp.int32),
            pltpu.VMEM((c, H), _F32),
            pltpu.VMEM_SHARED((NPAD, H), _F32),
            pltpu.SemaphoreType.DMA,
        ],
    )
    def k(u_hbm, p_hbm, dst_hbm, zu_hbm, ou_hbm, op_hbm,
          idx_v, idx2_v, d_v, acc, sem):
        ci = jax.lax.axis_index("c")
        si = jax.lax.axis_index("s")
        row0 = si * rps
        pltpu.sync_copy(zu_hbm.at[pl.ds(row0, rps)], acc.at[pl.ds(row0, rps)])
        plsc.subcore_barrier()

        def body(src_hbm):
            @pl.loop(0, nch)
            def _(j):
                base = si * per_s + j * c
                pltpu.sync_copy(dst_hbm.at[pl.ds(base, c)], idx_v)
                pltpu.sync_copy(src_hbm.at[pl.ds(base, c)], d_v)

                for sb in range(nsb):
                    for kk in list(range(0, cs - 15, 16)) + (
                            [cs - 16] if cs % 16 else []):
                        idx2_v[sb, pl.ds(kk, 16)] = (
                            idx_v[pl.ds(sb * cs + kk, 16)])

                cps = [pltpu.async_copy(
                    d_v.at[pl.ds(sb * cs, cs)],
                    acc.at[idx2_v.at[sb]], sem, add=True)
                    for sb in range(nsb)]
                for cp in cps:
                    cp.wait()

        @pl.when(ci == 0)
        def _():
            body(u_hbm)

        @pl.when(ci == 1)
        def _():
            body(p_hbm)

        plsc.subcore_barrier()

        @pl.when(ci == 0)
        def _():
            pltpu.sync_copy(acc.at[pl.ds(row0, rps)],
                            ou_hbm.at[pl.ds(row0, rps)])

        @pl.when(ci == 1)
        def _():
            pltpu.sync_copy(acc.at[pl.ds(row0, rps)],
                            op_hbm.at[pl.ds(row0, rps)])

    return k(u, p128, dst, zu)


def _r1(v):
    return v.reshape(1, -1)


def kernel(field, pos, bc_type, edge_index, face_normals, face_areas,
           face_type, params):
    src = edge_index[0].astype(jnp.int32)
    dst = edge_index[1].astype(jnp.int32)

    lane = jax.lax.broadcasted_iota(jnp.int32, (H, 16), 0) // 16
    head = jax.lax.broadcasted_iota(jnp.int32, (H, 16), 1)
    sel = jnp.where(lane == head, 0.25, 0.0).astype(_F32)
    exp16 = sel.T * 4.0

    ne = params['node_enc']
    dist_lo = jnp.clip(pos, -1.0, 1.0)
    dist_hi = jnp.clip(1.0 - pos, -1.0, 1.0)
    bcoh = (bc_type[:, None] == jnp.arange(5)[None, :]).astype(_F32)
    nf16 = jnp.concatenate(
        [field, dist_lo, dist_hi, bcoh, jnp.zeros((N, 3), _F32)], axis=1)
    wn1 = ne['linears'][0]['W']
    wn16 = jnp.concatenate(
        [wn1[:8], params['bc_embed'] @ wn1[8:16], jnp.zeros((3, H), _F32)], axis=0)
    x = _mlp16(nf16, wn16, _r1(ne['linears'][0]['b']),
               ne['linears'][1]['W'], _r1(ne['linears'][1]['b']),
               ne['linears'][2]['W'], _r1(ne['linears'][2]['b']),
               _r1(ne['ln']['g']), _r1(ne['ln']['b']))

    dx, dy = _sc_pos_delta(pos[:, 0], pos[:, 1], dst, src)
    delta = jnp.stack([dx, dy], axis=1)
    dist = jnp.maximum(
        jnp.sqrt(jnp.sum(delta * delta, axis=1, keepdims=True)), 1e-8)
    unit = delta / dist
    ftoh = (face_type[:, None] == jnp.arange(4)[None, :]).astype(_F32)
    ef16 = jnp.concatenate(
        [face_normals, face_areas[:, None], dist, unit, ftoh,
         jnp.zeros((E, 6), _F32)], axis=1)
    ee = params['edge_enc']
    we1 = ee['linears'][0]['W']
    we16 = jnp.concatenate(
        [we1[:6], params['ft_embed'] @ we1[6:10], jnp.zeros((6, H), _F32)], axis=0)
    e = _mlp16(ef16, we16, _r1(ee['linears'][0]['b']),
               ee['linears'][1]['W'], _r1(ee['linears'][1]['b']),
               ee['linears'][2]['W'], _r1(ee['linears'][2]['b']),
               _r1(ee['ln']['g']), _r1(ee['ln']['b']))

    zu = jnp.zeros((NPAD, H), _F32)
    expb128 = jnp.concatenate([exp16, jnp.zeros((H - 16, H), _F32)], axis=0)
    dec = params['dec']
    dec_ws = (dec['linears'][0]['W'], _r1(dec['linears'][0]['b']),
              dec['linears'][1]['W'], _r1(dec['linears'][1]['b']),
              jnp.pad(dec['linears'][2]['W'], ((0, 0), (0, H - 4))),
              _r1(jnp.pad(dec['linears'][2]['b'], (0, H - 4))))

    for b, bp in enumerate(params['blocks']):
        gxd, gxs = _sc_gather2(x, dst, src)
        em = bp['edge_mlp']
        wqk = jnp.block([[bp['W_Q'], jnp.zeros((H, H), _F32)],
                         [jnp.zeros((H, H), _F32), bp['W_K']]])
        emit_e = b == 0
        eouts = _edge_block(
            emit_e, gxd, gxs, e,
            em['linears'][0]['W'], _r1(em['linears'][0]['b']),
            em['linears'][1]['W'], _r1(em['linears'][1]['b']),
            em['linears'][2]['W'], _r1(em['linears'][2]['b']),
            _r1(em['ln']['g']), _r1(em['ln']['b']),
            wqk, bp['W_V'], sel, exp16)
        if emit_e:
            e_next, u, p128 = eouts
        else:
            u, p128 = eouts
            e_next = e
        accu, accp = _sc_scatter_add(u, p128, dst, zu)
        nm = bp['node_mlp']
        node_ws = (nm['linears'][0]['W'], _r1(nm['linears'][0]['b']),
                   nm['linears'][1]['W'], _r1(nm['linears'][1]['b']),
                   nm['linears'][2]['W'], _r1(nm['linears'][2]['b']),
                   _r1(nm['ln']['g']), _r1(nm['ln']['b']),
                   bp['W_O'], expb128)
        x = _node_block(x, accu, accp, node_ws,
                        dec_weights=dec_ws if b == 1 else None)
        e = e_next

    return x[:, :4]


# speedup vs baseline: 5.6639x; 1.0313x over previous
import dataclasses
import functools

import jax
import jax.numpy as jnp
from jax.experimental import pallas as pl
from jax.experimental.pallas import tpu as pltpu
from jax.experimental.pallas import tpu_sc as plsc

N = 10000
E = 160000
H = 128
NHEADS = 8
NPAD = 10240
NC, NS = 2, 16
NW = NC * NS
TT = 2000

_F32 = jnp.float32


def _ln(h, g, b):
    mu = jnp.mean(h, axis=1, keepdims=True)
    var = jnp.mean((h - mu) ** 2, axis=1, keepdims=True)
    return (h - mu) * jax.lax.rsqrt(var + 1e-5) * g + b


def _dot(a, b):
    return jnp.dot(a, b, preferred_element_type=_F32)


def _enc_body(xin, w1, b1, w2, b2, w3, b3, g, bt, out):
    h = jnp.maximum(_dot(xin[...], w1[...]) + b1[...], 0.0)
    h = jnp.maximum(_dot(h, w2[...]) + b2[...], 0.0)
    h = _dot(h, w3[...]) + b3[...]
    out[...] = _ln(h, g[...], bt[...])


def _mlp16(xin, w1, b1, w2, b2, w3, b3, g, bt):
    rows = xin.shape[0]
    grid = rows // TT
    wspec = lambda a: pl.BlockSpec(a.shape, lambda i: (0,) * a.ndim)
    return pl.pallas_call(
        _enc_body,
        grid=(grid,),
        in_specs=[pl.BlockSpec((TT, 16), lambda i: (i, 0))]
        + [wspec(a) for a in (w1, b1, w2, b2, w3, b3, g, bt)],
        out_specs=pl.BlockSpec((TT, H), lambda i: (i, 0)),
        out_shape=jax.ShapeDtypeStruct((rows, H), _F32),
    )(xin, w1, b1, w2, b2, w3, b3, g, bt)


def _edge_body(emit_e, gxd, gxs, e, w1, b1, w2, b2, w3, b3, g, bt, wqk, wv,
               sel, exp16, *outs):
    if emit_e:
        e1_out, u_out, p_out = outs
    else:
        u_out, p_out = outs
    xd = gxd[...]
    xs = gxs[...]
    ev = e[...]
    cat = jnp.concatenate([xd, xs, ev], axis=1)
    h = jnp.maximum(_dot(cat, w1[...]) + b1[...], 0.0)
    h = jnp.maximum(_dot(h, w2[...]) + b2[...], 0.0)
    h = _dot(h, w3[...]) + b3[...]
    e_new = _ln(h, g[...], bt[...])
    if emit_e:
        e1_out[...] = ev + e_new
    qk = _dot(jnp.concatenate([xd, xs], axis=1), wqk[...])
    prod = qk[:, :H] * qk[:, H:]
    p = jnp.exp(_dot(prod, sel[...]))
    v = _dot(e_new, wv[...])
    u_out[...] = _dot(p, exp16[...]) * v
    p_out[...] = jnp.concatenate(
        [p, jnp.zeros((p.shape[0], H - 16), _F32)], axis=1)


def _edge_block(emit_e, gxd, gxs, e, w1, b1, w2, b2, w3, b3, g, bt, wqk, wv,
                sel, exp16):
    ee = gxd.shape[0]
    grid = ee // TT
    wspec = lambda a: pl.BlockSpec(a.shape, lambda i: (0,) * a.ndim)
    ws = (w1, b1, w2, b2, w3, b3, g, bt, wqk, wv, sel, exp16)
    nout = 3 if emit_e else 2
    return pl.pallas_call(
        functools.partial(_edge_body, emit_e),
        grid=(grid,),
        in_specs=[pl.BlockSpec((TT, H), lambda i: (i, 0))] * 3
        + [wspec(a) for a in ws],
        out_specs=[pl.BlockSpec((TT, H), lambda i: (i, 0))] * nout,
        out_shape=[jax.ShapeDtypeStruct((ee, H), _F32)] * nout,
    )(gxd, gxs, e, *ws)


def _node_body(dec, x, accu1, accp1, accu2, accp2,
               w1, b1, w2, b2, w3, b3, g, bt, wo, expb, *rest):
    if dec:
        (wd1, bd1, wd2, bd2, wd3, bd3, out) = rest
    else:
        (out,) = rest
    xv = x[...]
    u = accu1[...] + accu2[...]
    pm = accp1[...] + accp2[...]
    rec = 1.0 / (pm + 1e-12)
    recb = _dot(rec, expb[...])
    msg = _dot(u * recb, wo[...])
    cat = jnp.concatenate([xv, msg], axis=1)
    h = jnp.maximum(_dot(cat, w1[...]) + b1[...], 0.0)
    h = jnp.maximum(_dot(h, w2[...]) + b2[...], 0.0)
    h = _dot(h, w3[...]) + b3[...]
    xn = xv + _ln(h, g[...], bt[...])
    if dec:
        h = jnp.maximum(_dot(xn, wd1[...]) + bd1[...], 0.0)
        h = jnp.maximum(_dot(h, wd2[...]) + bd2[...], 0.0)
        out[...] = _dot(h, wd3[...]) + bd3[...]
    else:
        out[...] = xn


def _node_block(x, accs, weights, dec_weights=None):
    grid = N // TT
    wspec = lambda a: pl.BlockSpec(a.shape, lambda i: (0,) * a.ndim)
    dec = dec_weights is not None
    ws = tuple(weights) + (tuple(dec_weights) if dec else ())
    return pl.pallas_call(
        functools.partial(_node_body, dec),
        grid=(grid,),
        in_specs=[pl.BlockSpec((TT, H), lambda i: (i, 0))] * 5
        + [wspec(a) for a in ws],
        out_specs=pl.BlockSpec((TT, H), lambda i: (i, 0)),
        out_shape=jax.ShapeDtypeStruct((N, H), _F32),
    )(x, *accs, *ws)


def _sc_gather2(table, idx_a, idx_b):
    d = table.shape[1]
    ee = idx_a.shape[0]
    ci = 200
    ng = ee // ci
    nbase, nrem = divmod(ng, NW)
    mesh = plsc.VectorSubcoreMesh(core_axis_name="c", subcore_axis_name="s")

    @functools.partial(
        pl.kernel,
        mesh=mesh,
        out_type=[jax.ShapeDtypeStruct((ee, d), _F32)] * 2,
        scratch_types=[
            pltpu.VMEM((ci,), jnp.int32),
            pltpu.VMEM((ci, d), _F32),
            pltpu.SemaphoreType.DMA,
        ],
    )
    def k(tab_hbm, ia_hbm, ib_hbm, oa_hbm, ob_hbm, idx_v, rows_v, sem):
        wid = jax.lax.axis_index("s") * NC + jax.lax.axis_index("c")
        nch = nbase + jnp.where(wid < nrem, 1, 0)

        @pl.loop(0, nch)
        def _(j):
            base = (wid + j * NW) * ci

            def one(i_hbm, o_hbm):
                pltpu.sync_copy(i_hbm.at[pl.ds(base, ci)], idx_v)
                pltpu.async_copy(tab_hbm.at[idx_v], rows_v, sem).wait()
                pltpu.sync_copy(rows_v, o_hbm.at[pl.ds(base, ci)])

            one(ia_hbm, oa_hbm)
            one(ib_hbm, ob_hbm)

    return k(table, idx_a, idx_b)


def _sc_pos_delta(px, py, dst, src):
    per_w = E // NW
    mesh = plsc.VectorSubcoreMesh(core_axis_name="c", subcore_axis_name="s")
    cp = pltpu.CompilerParams()
    if "needs_layout_passes" in pltpu.CompilerParams.__dataclass_fields__:
        cp = dataclasses.replace(cp, needs_layout_passes=False)

    @functools.partial(
        pl.kernel,
        mesh=mesh,
        compiler_params=cp,
        out_type=[jax.ShapeDtypeStruct((E,), _F32)] * 2,
        scratch_types=[
            pltpu.VMEM((N,), _F32),
            pltpu.VMEM((N,), _F32),
            pltpu.VMEM((per_w,), jnp.int32),
            pltpu.VMEM((per_w,), jnp.int32),
            pltpu.VMEM((per_w,), _F32),
            pltpu.VMEM((per_w,), _F32),
        ],
    )
    def k(px_hbm, py_hbm, dst_hbm, src_hbm, dx_hbm, dy_hbm,
          px_v, py_v, id_v, is_v, dx_v, dy_v):
        wid = jax.lax.axis_index("s") * NC + jax.lax.axis_index("c")
        base = wid * per_w
        pltpu.sync_copy(px_hbm, px_v)
        pltpu.sync_copy(py_hbm, py_v)
        pltpu.sync_copy(dst_hbm.at[pl.ds(base, per_w)], id_v)
        pltpu.sync_copy(src_hbm.at[pl.ds(base, per_w)], is_v)

        def body(kk):
            d = id_v[pl.ds(kk, 16)]
            s = is_v[pl.ds(kk, 16)]
            dx_v[pl.ds(kk, 16)] = (plsc.load_gather(px_v, [d])
                                   - plsc.load_gather(px_v, [s]))
            dy_v[pl.ds(kk, 16)] = (plsc.load_gather(py_v, [d])
                                   - plsc.load_gather(py_v, [s]))

        nfull = per_w // 16 * 16

        @pl.loop(0, nfull, step=16)
        def _(kk):
            body(kk)

        if per_w % 16:
            body(per_w - 16)

        pltpu.sync_copy(dx_v, dx_hbm.at[pl.ds(base, per_w)])
        pltpu.sync_copy(dy_v, dy_hbm.at[pl.ds(base, per_w)])

    return k(px, py, dst, src)


def _sc_scatter_add(u, p128, dst, zu):
    c = 200
    cs = 40
    nsb = c // cs
    ee = dst.shape[0]
    ng = ee // c
    nbase, nrem = divmod(ng, NS)
    rps = NPAD // NS
    mesh = plsc.VectorSubcoreMesh(core_axis_name="c", subcore_axis_name="s")

    @functools.partial(
        pl.kernel,
        mesh=mesh,
        out_type=[
            jax.ShapeDtypeStruct((NPAD, H), _F32),
            jax.ShapeDtypeStruct((NPAD, H), _F32),
        ],
        scratch_types=[
            pltpu.VMEM((c,), jnp.int32),
            pltpu.VMEM((nsb, cs), jnp.int32),
            pltpu.VMEM((c, H), _F32),
            pltpu.VMEM_SHARED((NPAD, H), _F32),
            pltpu.SemaphoreType.DMA,
        ],
    )
    def k(u_hbm, p_hbm, dst_hbm, zu_hbm, ou_hbm, op_hbm,
          idx_v, idx2_v, d_v, acc, sem):
        ci = jax.lax.axis_index("c")
        si = jax.lax.axis_index("s")
        row0 = si * rps
        pltpu.sync_copy(zu_hbm.at[pl.ds(row0, rps)], acc.at[pl.ds(row0, rps)])
        plsc.subcore_barrier()

        def body(src_hbm):
            nch = nbase + jnp.where(si < nrem, 1, 0)

            @pl.loop(0, nch)
            def _(j):
                base = (si + j * NS) * c
                pltpu.sync_copy(dst_hbm.at[pl.ds(base, c)], idx_v)
                pltpu.sync_copy(src_hbm.at[pl.ds(base, c)], d_v)

                for sb in range(nsb):
                    for kk in list(range(0, cs - 15, 16)) + (
                            [cs - 16] if cs % 16 else []):
                        idx2_v[sb, pl.ds(kk, 16)] = (
                            idx_v[pl.ds(sb * cs + kk, 16)])

                cps = [pltpu.async_copy(
                    d_v.at[pl.ds(sb * cs, cs)],
                    acc.at[idx2_v.at[sb]], sem, add=True)
                    for sb in range(nsb)]
                for cp in cps:
                    cp.wait()

        @pl.when(ci == 0)
        def _():
            body(u_hbm)

        @pl.when(ci == 1)
        def _():
            body(p_hbm)

        plsc.subcore_barrier()

        @pl.when(ci == 0)
        def _():
            pltpu.sync_copy(acc.at[pl.ds(row0, rps)],
                            ou_hbm.at[pl.ds(row0, rps)])

        @pl.when(ci == 1)
        def _():
            pltpu.sync_copy(acc.at[pl.ds(row0, rps)],
                            op_hbm.at[pl.ds(row0, rps)])

    return k(u, p128, dst, zu)


def _r1(v):
    return v.reshape(1, -1)


def kernel(field, pos, bc_type, edge_index, face_normals, face_areas,
           face_type, params):
    src = edge_index[0].astype(jnp.int32)
    dst = edge_index[1].astype(jnp.int32)

    lane = jax.lax.broadcasted_iota(jnp.int32, (H, 16), 0) // 16
    head = jax.lax.broadcasted_iota(jnp.int32, (H, 16), 1)
    sel = jnp.where(lane == head, 0.25, 0.0).astype(_F32)
    exp16 = sel.T * 4.0

    ne = params['node_enc']
    dist_lo = jnp.clip(pos, -1.0, 1.0)
    dist_hi = jnp.clip(1.0 - pos, -1.0, 1.0)
    bcoh = (bc_type[:, None] == jnp.arange(5)[None, :]).astype(_F32)
    nf16 = jnp.concatenate(
        [field, dist_lo, dist_hi, bcoh, jnp.zeros((N, 3), _F32)], axis=1)
    wn1 = ne['linears'][0]['W']
    wn16 = jnp.concatenate(
        [wn1[:8], params['bc_embed'] @ wn1[8:16], jnp.zeros((3, H), _F32)], axis=0)
    x = _mlp16(nf16, wn16, _r1(ne['linears'][0]['b']),
               ne['linears'][1]['W'], _r1(ne['linears'][1]['b']),
               ne['linears'][2]['W'], _r1(ne['linears'][2]['b']),
               _r1(ne['ln']['g']), _r1(ne['ln']['b']))

    dx, dy = _sc_pos_delta(pos[:, 0], pos[:, 1], dst, src)
    delta = jnp.stack([dx, dy], axis=1)
    dist = jnp.maximum(
        jnp.sqrt(jnp.sum(delta * delta, axis=1, keepdims=True)), 1e-8)
    unit = delta / dist
    ftoh = (face_type[:, None] == jnp.arange(4)[None, :]).astype(_F32)
    ef16 = jnp.concatenate(
        [face_normals, face_areas[:, None], dist, unit, ftoh,
         jnp.zeros((E, 6), _F32)], axis=1)
    ee = params['edge_enc']
    we1 = ee['linears'][0]['W']
    we16 = jnp.concatenate(
        [we1[:6], params['ft_embed'] @ we1[6:10], jnp.zeros((6, H), _F32)], axis=0)
    e = _mlp16(ef16, we16, _r1(ee['linears'][0]['b']),
               ee['linears'][1]['W'], _r1(ee['linears'][1]['b']),
               ee['linears'][2]['W'], _r1(ee['linears'][2]['b']),
               _r1(ee['ln']['g']), _r1(ee['ln']['b']))

    zu = jnp.zeros((NPAD, H), _F32)
    expb128 = jnp.concatenate([exp16, jnp.zeros((H - 16, H), _F32)], axis=0)
    dec = params['dec']
    dec_ws = (dec['linears'][0]['W'], _r1(dec['linears'][0]['b']),
              dec['linears'][1]['W'], _r1(dec['linears'][1]['b']),
              jnp.pad(dec['linears'][2]['W'], ((0, 0), (0, H - 4))),
              _r1(jnp.pad(dec['linears'][2]['b'], (0, H - 4))))

    ec = E // 2
    dst_h = (dst[:ec], dst[ec:])
    src_h = (src[:ec], src[ec:])
    e_h = [e[:ec], e[ec:]]

    for b, bp in enumerate(params['blocks']):
        em = bp['edge_mlp']
        wqk = jnp.block([[bp['W_Q'], jnp.zeros((H, H), _F32)],
                         [jnp.zeros((H, H), _F32), bp['W_K']]])
        emit_e = b == 0
        accs = []
        e_next_h = []
        for hh in range(2):
            gxd, gxs = _sc_gather2(x, dst_h[hh], src_h[hh])
            eouts = _edge_block(
                emit_e, gxd, gxs, e_h[hh],
                em['linears'][0]['W'], _r1(em['linears'][0]['b']),
                em['linears'][1]['W'], _r1(em['linears'][1]['b']),
                em['linears'][2]['W'], _r1(em['linears'][2]['b']),
                _r1(em['ln']['g']), _r1(em['ln']['b']),
                wqk, bp['W_V'], sel, exp16)
            if emit_e:
                e_next, u, p128 = eouts
                e_next_h.append(e_next)
            else:
                u, p128 = eouts
            accs.extend(_sc_scatter_add(u, p128, dst_h[hh], zu))
        nm = bp['node_mlp']
        node_ws = (nm['linears'][0]['W'], _r1(nm['linears'][0]['b']),
                   nm['linears'][1]['W'], _r1(nm['linears'][1]['b']),
                   nm['linears'][2]['W'], _r1(nm['linears'][2]['b']),
                   _r1(nm['ln']['g']), _r1(nm['ln']['b']),
                   bp['W_O'], expb128)
        x = _node_block(x, accs, node_ws,
                        dec_weights=dec_ws if b == 1 else None)
        if emit_e:
            e_h = e_next_h

    return x[:, :4]


# speedup vs baseline: 5.8495x; 1.0328x over previous
import dataclasses
import functools

import jax
import jax.numpy as jnp
from jax.experimental import pallas as pl
from jax.experimental.pallas import tpu as pltpu
from jax.experimental.pallas import tpu_sc as plsc

N = 10000
E = 160000
H = 128
NHEADS = 8
NPAD = 10240
NC, NS = 2, 16
NW = NC * NS
TT = 2000

_F32 = jnp.float32


def _ln(h, g, b):
    mu = jnp.mean(h, axis=1, keepdims=True)
    var = jnp.mean((h - mu) ** 2, axis=1, keepdims=True)
    return (h - mu) * jax.lax.rsqrt(var + 1e-5) * g + b


def _dot(a, b):
    return jnp.dot(a, b, preferred_element_type=_F32)


def _enc_body(xin, w1, b1, w2, b2, w3, b3, g, bt, out):
    h = jnp.maximum(_dot(xin[...], w1[...]) + b1[...], 0.0)
    h = jnp.maximum(_dot(h, w2[...]) + b2[...], 0.0)
    h = _dot(h, w3[...]) + b3[...]
    out[...] = _ln(h, g[...], bt[...])


def _mlp16(xin, w1, b1, w2, b2, w3, b3, g, bt):
    rows = xin.shape[0]
    grid = rows // TT
    wspec = lambda a: pl.BlockSpec(a.shape, lambda i: (0,) * a.ndim)
    return pl.pallas_call(
        _enc_body,
        grid=(grid,),
        in_specs=[pl.BlockSpec((TT, 16), lambda i: (i, 0))]
        + [wspec(a) for a in (w1, b1, w2, b2, w3, b3, g, bt)],
        out_specs=pl.BlockSpec((TT, H), lambda i: (i, 0)),
        out_shape=jax.ShapeDtypeStruct((rows, H), _F32),
    )(xin, w1, b1, w2, b2, w3, b3, g, bt)


def _edge_body(emit_e, gxd, gxs, e, w1, b1, w2, b2, w3, b3, g, bt, wqk, wv,
               sel, exp16, *outs):
    if emit_e:
        e1_out, u_out, p_out = outs
    else:
        u_out, p_out = outs
    xd = gxd[...]
    xs = gxs[...]
    ev = e[...]
    cat = jnp.concatenate([xd, xs, ev], axis=1)
    h = jnp.maximum(_dot(cat, w1[...]) + b1[...], 0.0)
    h = jnp.maximum(_dot(h, w2[...]) + b2[...], 0.0)
    h = _dot(h, w3[...]) + b3[...]
    e_new = _ln(h, g[...], bt[...])
    if emit_e:
        e1_out[...] = ev + e_new
    qk = _dot(jnp.concatenate([xd, xs], axis=1), wqk[...])
    prod = qk[:, :H] * qk[:, H:]
    p = jnp.exp(_dot(prod, sel[...]))
    v = _dot(e_new, wv[...])
    u_out[...] = _dot(p, exp16[...]) * v
    p_out[...] = jnp.concatenate(
        [p, jnp.zeros((p.shape[0], H - 16), _F32)], axis=1)


def _edge_block(emit_e, gxd, gxs, e, w1, b1, w2, b2, w3, b3, g, bt, wqk, wv,
                sel, exp16):
    ee = gxd.shape[0]
    grid = ee // TT
    wspec = lambda a: pl.BlockSpec(a.shape, lambda i: (0,) * a.ndim)
    ws = (w1, b1, w2, b2, w3, b3, g, bt, wqk, wv, sel, exp16)
    nout = 3 if emit_e else 2
    return pl.pallas_call(
        functools.partial(_edge_body, emit_e),
        grid=(grid,),
        in_specs=[pl.BlockSpec((TT, H), lambda i: (i, 0))] * 3
        + [wspec(a) for a in ws],
        out_specs=[pl.BlockSpec((TT, H), lambda i: (i, 0))] * nout,
        out_shape=[jax.ShapeDtypeStruct((ee, H), _F32)] * nout,
    )(gxd, gxs, e, *ws)


def _node_body(dec, x, accu1, accp1, accu2, accp2,
               w1, b1, w2, b2, w3, b3, g, bt, wo, expb, *rest):
    if dec:
        (wd1, bd1, wd2, bd2, wd3, bd3, out) = rest
    else:
        (out,) = rest
    xv = x[...]
    u = accu1[...] + accu2[...]
    pm = accp1[...] + accp2[...]
    rec = 1.0 / (pm + 1e-12)
    recb = _dot(rec, expb[...])
    msg = _dot(u * recb, wo[...])
    cat = jnp.concatenate([xv, msg], axis=1)
    h = jnp.maximum(_dot(cat, w1[...]) + b1[...], 0.0)
    h = jnp.maximum(_dot(h, w2[...]) + b2[...], 0.0)
    h = _dot(h, w3[...]) + b3[...]
    xn = xv + _ln(h, g[...], bt[...])
    if dec:
        h = jnp.maximum(_dot(xn, wd1[...]) + bd1[...], 0.0)
        h = jnp.maximum(_dot(h, wd2[...]) + bd2[...], 0.0)
        out[...] = _dot(h, wd3[...]) + bd3[...]
    else:
        out[...] = xn


def _node_block(x, accs, weights, dec_weights=None):
    grid = N // TT
    wspec = lambda a: pl.BlockSpec(a.shape, lambda i: (0,) * a.ndim)
    dec = dec_weights is not None
    ws = tuple(weights) + (tuple(dec_weights) if dec else ())
    return pl.pallas_call(
        functools.partial(_node_body, dec),
        grid=(grid,),
        in_specs=[pl.BlockSpec((TT, H), lambda i: (i, 0))] * 5
        + [wspec(a) for a in ws],
        out_specs=pl.BlockSpec((TT, H), lambda i: (i, 0)),
        out_shape=jax.ShapeDtypeStruct((N, H), _F32),
    )(x, *accs, *ws)


def _sc_gather2(table, idx_a, idx_b):
    d = table.shape[1]
    ee = idx_a.shape[0]
    ci = 200
    ng = ee // ci
    nbase, nrem = divmod(ng, NW)
    mesh = plsc.VectorSubcoreMesh(core_axis_name="c", subcore_axis_name="s")

    @functools.partial(
        pl.kernel,
        mesh=mesh,
        out_type=[jax.ShapeDtypeStruct((ee, d), _F32)] * 2,
        scratch_types=[
            pltpu.VMEM((ci,), jnp.int32),
            pltpu.VMEM((ci,), jnp.int32),
            pltpu.VMEM((ci, d), _F32),
            pltpu.VMEM((ci, d), _F32),
            pltpu.SemaphoreType.DMA,
            pltpu.SemaphoreType.DMA,
        ],
    )
    def k(tab_hbm, ia_hbm, ib_hbm, oa_hbm, ob_hbm,
          ia_v, ib_v, ra_v, rb_v, semg, semw):
        wid = jax.lax.axis_index("s") * NC + jax.lax.axis_index("c")
        nch = nbase + jnp.where(wid < nrem, 1, 0)

        @pl.loop(0, nch)
        def _(j):
            base = (wid + j * NW) * ci
            cia = pltpu.async_copy(ia_hbm.at[pl.ds(base, ci)], ia_v, semg)
            cib = pltpu.async_copy(ib_hbm.at[pl.ds(base, ci)], ib_v, semg)
            cia.wait()
            cib.wait()

            @pl.when(j > 0)
            def _():
                pbase = (wid + (j - 1) * NW) * ci
                pltpu.make_async_copy(
                    ra_v, oa_hbm.at[pl.ds(pbase, ci)], semw).wait()
                pltpu.make_async_copy(
                    rb_v, ob_hbm.at[pl.ds(pbase, ci)], semw).wait()

            cga = pltpu.async_copy(tab_hbm.at[ia_v], ra_v, semg)
            cgb = pltpu.async_copy(tab_hbm.at[ib_v], rb_v, semg)
            cga.wait()
            cgb.wait()
            pltpu.async_copy(ra_v, oa_hbm.at[pl.ds(base, ci)], semw)
            pltpu.async_copy(rb_v, ob_hbm.at[pl.ds(base, ci)], semw)

        lbase = (wid + (nch - 1) * NW) * ci
        pltpu.make_async_copy(ra_v, oa_hbm.at[pl.ds(lbase, ci)], semw).wait()
        pltpu.make_async_copy(rb_v, ob_hbm.at[pl.ds(lbase, ci)], semw).wait()

    return k(table, idx_a, idx_b)


def _sc_pos_delta(px, py, dst, src):
    per_w = E // NW
    mesh = plsc.VectorSubcoreMesh(core_axis_name="c", subcore_axis_name="s")
    cp = pltpu.CompilerParams()
    if "needs_layout_passes" in pltpu.CompilerParams.__dataclass_fields__:
        cp = dataclasses.replace(cp, needs_layout_passes=False)

    @functools.partial(
        pl.kernel,
        mesh=mesh,
        compiler_params=cp,
        out_type=[jax.ShapeDtypeStruct((E,), _F32)] * 2,
        scratch_types=[
            pltpu.VMEM((N,), _F32),
            pltpu.VMEM((N,), _F32),
            pltpu.VMEM((per_w,), jnp.int32),
            pltpu.VMEM((per_w,), jnp.int32),
            pltpu.VMEM((per_w,), _F32),
            pltpu.VMEM((per_w,), _F32),
        ],
    )
    def k(px_hbm, py_hbm, dst_hbm, src_hbm, dx_hbm, dy_hbm,
          px_v, py_v, id_v, is_v, dx_v, dy_v):
        wid = jax.lax.axis_index("s") * NC + jax.lax.axis_index("c")
        base = wid * per_w
        pltpu.sync_copy(px_hbm, px_v)
        pltpu.sync_copy(py_hbm, py_v)
        pltpu.sync_copy(dst_hbm.at[pl.ds(base, per_w)], id_v)
        pltpu.sync_copy(src_hbm.at[pl.ds(base, per_w)], is_v)

        def body(kk):
            d = id_v[pl.ds(kk, 16)]
            s = is_v[pl.ds(kk, 16)]
            dx_v[pl.ds(kk, 16)] = (plsc.load_gather(px_v, [d])
                                   - plsc.load_gather(px_v, [s]))
            dy_v[pl.ds(kk, 16)] = (plsc.load_gather(py_v, [d])
                                   - plsc.load_gather(py_v, [s]))

        nfull = per_w // 16 * 16

        @pl.loop(0, nfull, step=16)
        def _(kk):
            body(kk)

        if per_w % 16:
            body(per_w - 16)

        pltpu.sync_copy(dx_v, dx_hbm.at[pl.ds(base, per_w)])
        pltpu.sync_copy(dy_v, dy_hbm.at[pl.ds(base, per_w)])

    return k(px, py, dst, src)


def _sc_scatter_add(u, p128, dst, zu):
    c = 200
    cs = 40
    nsb = c // cs
    ee = dst.shape[0]
    ng = ee // c
    nbase, nrem = divmod(ng, NS)
    rps = NPAD // NS
    mesh = plsc.VectorSubcoreMesh(core_axis_name="c", subcore_axis_name="s")

    @functools.partial(
        pl.kernel,
        mesh=mesh,
        out_type=[
            jax.ShapeDtypeStruct((NPAD, H), _F32),
            jax.ShapeDtypeStruct((NPAD, H), _F32),
        ],
        scratch_types=[
            pltpu.VMEM((c,), jnp.int32),
            pltpu.VMEM((nsb, cs), jnp.int32),
            pltpu.VMEM((c, H), _F32),
            pltpu.VMEM_SHARED((NPAD, H), _F32),
            pltpu.SemaphoreType.DMA,
        ],
    )
    def k(u_hbm, p_hbm, dst_hbm, zu_hbm, ou_hbm, op_hbm,
          idx_v, idx2_v, d_v, acc, sem):
        ci = jax.lax.axis_index("c")
        si = jax.lax.axis_index("s")
        row0 = si * rps
        pltpu.sync_copy(zu_hbm.at[pl.ds(row0, rps)], acc.at[pl.ds(row0, rps)])
        plsc.subcore_barrier()

        def body(src_hbm):
            nch = nbase + jnp.where(si < nrem, 1, 0)

            @pl.loop(0, nch)
            def _(j):
                base = (si + j * NS) * c
                pltpu.sync_copy(dst_hbm.at[pl.ds(base, c)], idx_v)
                pltpu.sync_copy(src_hbm.at[pl.ds(base, c)], d_v)

                for sb in range(nsb):
                    for kk in list(range(0, cs - 15, 16)) + (
                            [cs - 16] if cs % 16 else []):
                        idx2_v[sb, pl.ds(kk, 16)] = (
                            idx_v[pl.ds(sb * cs + kk, 16)])

                cps = [pltpu.async_copy(
                    d_v.at[pl.ds(sb * cs, cs)],
                    acc.at[idx2_v.at[sb]], sem, add=True)
                    for sb in range(nsb)]
                for cp in cps:
                    cp.wait()

        @pl.when(ci == 0)
        def _():
            body(u_hbm)

        @pl.when(ci == 1)
        def _():
            body(p_hbm)

        plsc.subcore_barrier()

        @pl.when(ci == 0)
        def _():
            pltpu.sync_copy(acc.at[pl.ds(row0, rps)],
                            ou_hbm.at[pl.ds(row0, rps)])

        @pl.when(ci == 1)
        def _():
            pltpu.sync_copy(acc.at[pl.ds(row0, rps)],
                            op_hbm.at[pl.ds(row0, rps)])

    return k(u, p128, dst, zu)


def _r1(v):
    return v.reshape(1, -1)


def kernel(field, pos, bc_type, edge_index, face_normals, face_areas,
           face_type, params):
    src = edge_index[0].astype(jnp.int32)
    dst = edge_index[1].astype(jnp.int32)

    lane = jax.lax.broadcasted_iota(jnp.int32, (H, 16), 0) // 16
    head = jax.lax.broadcasted_iota(jnp.int32, (H, 16), 1)
    sel = jnp.where(lane == head, 0.25, 0.0).astype(_F32)
    exp16 = sel.T * 4.0

    ne = params['node_enc']
    dist_lo = jnp.clip(pos, -1.0, 1.0)
    dist_hi = jnp.clip(1.0 - pos, -1.0, 1.0)
    bcoh = (bc_type[:, None] == jnp.arange(5)[None, :]).astype(_F32)
    nf16 = jnp.concatenate(
        [field, dist_lo, dist_hi, bcoh, jnp.zeros((N, 3), _F32)], axis=1)
    wn1 = ne['linears'][0]['W']
    wn16 = jnp.concatenate(
        [wn1[:8], params['bc_embed'] @ wn1[8:16], jnp.zeros((3, H), _F32)], axis=0)
    x = _mlp16(nf16, wn16, _r1(ne['linears'][0]['b']),
               ne['linears'][1]['W'], _r1(ne['linears'][1]['b']),
               ne['linears'][2]['W'], _r1(ne['linears'][2]['b']),
               _r1(ne['ln']['g']), _r1(ne['ln']['b']))

    dx, dy = _sc_pos_delta(pos[:, 0], pos[:, 1], dst, src)
    delta = jnp.stack([dx, dy], axis=1)
    dist = jnp.maximum(
        jnp.sqrt(jnp.sum(delta * delta, axis=1, keepdims=True)), 1e-8)
    unit = delta / dist
    ftoh = (face_type[:, None] == jnp.arange(4)[None, :]).astype(_F32)
    ef16 = jnp.concatenate(
        [face_normals, face_areas[:, None], dist, unit, ftoh,
         jnp.zeros((E, 6), _F32)], axis=1)
    ee = params['edge_enc']
    we1 = ee['linears'][0]['W']
    we16 = jnp.concatenate(
        [we1[:6], params['ft_embed'] @ we1[6:10], jnp.zeros((6, H), _F32)], axis=0)
    e = _mlp16(ef16, we16, _r1(ee['linears'][0]['b']),
               ee['linears'][1]['W'], _r1(ee['linears'][1]['b']),
               ee['linears'][2]['W'], _r1(ee['linears'][2]['b']),
               _r1(ee['ln']['g']), _r1(ee['ln']['b']))

    zu = jnp.zeros((NPAD, H), _F32)
    expb128 = jnp.concatenate([exp16, jnp.zeros((H - 16, H), _F32)], axis=0)
    dec = params['dec']
    dec_ws = (dec['linears'][0]['W'], _r1(dec['linears'][0]['b']),
              dec['linears'][1]['W'], _r1(dec['linears'][1]['b']),
              jnp.pad(dec['linears'][2]['W'], ((0, 0), (0, H - 4))),
              _r1(jnp.pad(dec['linears'][2]['b'], (0, H - 4))))

    ec = E // 2
    dst_h = (dst[:ec], dst[ec:])
    src_h = (src[:ec], src[ec:])
    e_h = [e[:ec], e[ec:]]

    for b, bp in enumerate(params['blocks']):
        em = bp['edge_mlp']
        wqk = jnp.block([[bp['W_Q'], jnp.zeros((H, H), _F32)],
                         [jnp.zeros((H, H), _F32), bp['W_K']]])
        emit_e = b == 0
        accs = []
        e_next_h = []
        for hh in range(2):
            gxd, gxs = _sc_gather2(x, dst_h[hh], src_h[hh])
            eouts = _edge_block(
                emit_e, gxd, gxs, e_h[hh],
                em['linears'][0]['W'], _r1(em['linears'][0]['b']),
                em['linears'][1]['W'], _r1(em['linears'][1]['b']),
                em['linears'][2]['W'], _r1(em['linears'][2]['b']),
                _r1(em['ln']['g']), _r1(em['ln']['b']),
                wqk, bp['W_V'], sel, exp16)
            if emit_e:
                e_next, u, p128 = eouts
                e_next_h.append(e_next)
            else:
                u, p128 = eouts
            accs.extend(_sc_scatter_add(u, p128, dst_h[hh], zu))
        nm = bp['node_mlp']
        node_ws = (nm['linears'][0]['W'], _r1(nm['linears'][0]['b']),
                   nm['linears'][1]['W'], _r1(nm['linears'][1]['b']),
                   nm['linears'][2]['W'], _r1(nm['linears'][2]['b']),
                   _r1(nm['ln']['g']), _r1(nm['ln']['b']),
                   bp['W_O'], expb128)
        x = _node_block(x, accs, node_ws,
                        dec_weights=dec_ws if b == 1 else None)
        if emit_e:
            e_h = e_next_h

    return x[:, :4]


# speedup vs baseline: 5.8611x; 1.0020x over previous
import dataclasses
import functools

import jax
import jax.numpy as jnp
from jax.experimental import pallas as pl
from jax.experimental.pallas import tpu as pltpu
from jax.experimental.pallas import tpu_sc as plsc

N = 10000
E = 160000
H = 128
NHEADS = 8
NPAD = 10240
NC, NS = 2, 16
NW = NC * NS
TT = 2000

_F32 = jnp.float32


def _ln(h, g, b):
    mu = jnp.mean(h, axis=1, keepdims=True)
    var = jnp.mean((h - mu) ** 2, axis=1, keepdims=True)
    return (h - mu) * jax.lax.rsqrt(var + 1e-5) * g + b


def _dot(a, b):
    return jnp.dot(a, b, preferred_element_type=_F32)


def _enc_body(xin, w1, b1, w2, b2, w3, b3, g, bt, out):
    h = jnp.maximum(_dot(xin[...], w1[...]) + b1[...], 0.0)
    h = jnp.maximum(_dot(h, w2[...]) + b2[...], 0.0)
    h = _dot(h, w3[...]) + b3[...]
    out[...] = _ln(h, g[...], bt[...])


def _mlp16(xin, w1, b1, w2, b2, w3, b3, g, bt):
    rows = xin.shape[0]
    grid = rows // TT
    wspec = lambda a: pl.BlockSpec(a.shape, lambda i: (0,) * a.ndim)
    return pl.pallas_call(
        _enc_body,
        grid=(grid,),
        in_specs=[pl.BlockSpec((TT, 16), lambda i: (i, 0))]
        + [wspec(a) for a in (w1, b1, w2, b2, w3, b3, g, bt)],
        out_specs=pl.BlockSpec((TT, H), lambda i: (i, 0)),
        out_shape=jax.ShapeDtypeStruct((rows, H), _F32),
    )(xin, w1, b1, w2, b2, w3, b3, g, bt)


def _edge_body(emit_e, gxd, gxs, e, w1, b1, w2, b2, w3, b3, g, bt, wqk, wv,
               sel, exp16, *outs):
    if emit_e:
        e1_out, u_out, p_out = outs
    else:
        u_out, p_out = outs
    xd = gxd[...]
    xs = gxs[...]
    ev = e[...]
    cat = jnp.concatenate([xd, xs, ev], axis=1)
    h = jnp.maximum(_dot(cat, w1[...]) + b1[...], 0.0)
    h = jnp.maximum(_dot(h, w2[...]) + b2[...], 0.0)
    h = _dot(h, w3[...]) + b3[...]
    e_new = _ln(h, g[...], bt[...])
    if emit_e:
        e1_out[...] = ev + e_new
    qk = _dot(jnp.concatenate([xd, xs], axis=1), wqk[...])
    prod = qk[:, :H] * qk[:, H:]
    p = jnp.exp(_dot(prod, sel[...]))
    v = _dot(e_new, wv[...])
    u_out[...] = _dot(p, exp16[...]) * v
    p_out[...] = jnp.concatenate(
        [p, jnp.zeros((p.shape[0], H - 16), _F32)], axis=1)


def _edge_block(emit_e, gxd, gxs, e, w1, b1, w2, b2, w3, b3, g, bt, wqk, wv,
                sel, exp16):
    ee = gxd.shape[0]
    grid = ee // TT
    wspec = lambda a: pl.BlockSpec(a.shape, lambda i: (0,) * a.ndim)
    ws = (w1, b1, w2, b2, w3, b3, g, bt, wqk, wv, sel, exp16)
    nout = 3 if emit_e else 2
    return pl.pallas_call(
        functools.partial(_edge_body, emit_e),
        grid=(grid,),
        in_specs=[pl.BlockSpec((TT, H), lambda i: (i, 0))] * 3
        + [wspec(a) for a in ws],
        out_specs=[pl.BlockSpec((TT, H), lambda i: (i, 0))] * nout,
        out_shape=[jax.ShapeDtypeStruct((ee, H), _F32)] * nout,
    )(gxd, gxs, e, *ws)


def _node_body(dec, x, accu1, accp1, accu2, accp2,
               w1, b1, w2, b2, w3, b3, g, bt, wo, expb, *rest):
    if dec:
        (wd1, bd1, wd2, bd2, wd3, bd3, out) = rest
    else:
        (out,) = rest
    xv = x[...]
    u = accu1[...] + accu2[...]
    pm = accp1[...] + accp2[...]
    rec = 1.0 / (pm + 1e-12)
    recb = _dot(rec, expb[...])
    msg = _dot(u * recb, wo[...])
    cat = jnp.concatenate([xv, msg], axis=1)
    h = jnp.maximum(_dot(cat, w1[...]) + b1[...], 0.0)
    h = jnp.maximum(_dot(h, w2[...]) + b2[...], 0.0)
    h = _dot(h, w3[...]) + b3[...]
    xn = xv + _ln(h, g[...], bt[...])
    if dec:
        h = jnp.maximum(_dot(xn, wd1[...]) + bd1[...], 0.0)
        h = jnp.maximum(_dot(h, wd2[...]) + bd2[...], 0.0)
        out[...] = _dot(h, wd3[...]) + bd3[...]
    else:
        out[...] = xn


def _node_block(x, accs, weights, dec_weights=None):
    grid = N // TT
    wspec = lambda a: pl.BlockSpec(a.shape, lambda i: (0,) * a.ndim)
    dec = dec_weights is not None
    ws = tuple(weights) + (tuple(dec_weights) if dec else ())
    return pl.pallas_call(
        functools.partial(_node_body, dec),
        grid=(grid,),
        in_specs=[pl.BlockSpec((TT, H), lambda i: (i, 0))] * 5
        + [wspec(a) for a in ws],
        out_specs=pl.BlockSpec((TT, H), lambda i: (i, 0)),
        out_shape=jax.ShapeDtypeStruct((N, H), _F32),
    )(x, *accs, *ws)


def _sc_gather2(table, idx_a, idx_b):
    d = table.shape[1]
    ee = idx_a.shape[0]
    ci = 400
    ng = ee // ci
    nbase, nrem = divmod(ng, NW)
    mesh = plsc.VectorSubcoreMesh(core_axis_name="c", subcore_axis_name="s")

    @functools.partial(
        pl.kernel,
        mesh=mesh,
        out_type=[jax.ShapeDtypeStruct((ee, d), _F32)] * 2,
        scratch_types=[
            pltpu.VMEM((ci,), jnp.int32),
            pltpu.VMEM((ci,), jnp.int32),
            pltpu.VMEM((ci, d), _F32),
            pltpu.VMEM((ci, d), _F32),
            pltpu.SemaphoreType.DMA,
            pltpu.SemaphoreType.DMA,
        ],
    )
    def k(tab_hbm, ia_hbm, ib_hbm, oa_hbm, ob_hbm,
          ia_v, ib_v, ra_v, rb_v, semg, semw):
        wid = jax.lax.axis_index("s") * NC + jax.lax.axis_index("c")
        nch = nbase + jnp.where(wid < nrem, 1, 0)

        @pl.loop(0, nch)
        def _(j):
            base = (wid + j * NW) * ci
            cia = pltpu.async_copy(ia_hbm.at[pl.ds(base, ci)], ia_v, semg)
            cib = pltpu.async_copy(ib_hbm.at[pl.ds(base, ci)], ib_v, semg)
            cia.wait()
            cib.wait()

            @pl.when(j > 0)
            def _():
                pbase = (wid + (j - 1) * NW) * ci
                pltpu.make_async_copy(
                    ra_v, oa_hbm.at[pl.ds(pbase, ci)], semw).wait()
                pltpu.make_async_copy(
                    rb_v, ob_hbm.at[pl.ds(pbase, ci)], semw).wait()

            cga = pltpu.async_copy(tab_hbm.at[ia_v], ra_v, semg)
            cgb = pltpu.async_copy(tab_hbm.at[ib_v], rb_v, semg)
            cga.wait()
            cgb.wait()
            pltpu.async_copy(ra_v, oa_hbm.at[pl.ds(base, ci)], semw)
            pltpu.async_copy(rb_v, ob_hbm.at[pl.ds(base, ci)], semw)

        lbase = (wid + (nch - 1) * NW) * ci
        pltpu.make_async_copy(ra_v, oa_hbm.at[pl.ds(lbase, ci)], semw).wait()
        pltpu.make_async_copy(rb_v, ob_hbm.at[pl.ds(lbase, ci)], semw).wait()

    return k(table, idx_a, idx_b)


def _sc_pos_delta(px, py, dst, src):
    per_w = E // NW
    mesh = plsc.VectorSubcoreMesh(core_axis_name="c", subcore_axis_name="s")
    cp = pltpu.CompilerParams()
    if "needs_layout_passes" in pltpu.CompilerParams.__dataclass_fields__:
        cp = dataclasses.replace(cp, needs_layout_passes=False)

    @functools.partial(
        pl.kernel,
        mesh=mesh,
        compiler_params=cp,
        out_type=[jax.ShapeDtypeStruct((E,), _F32)] * 2,
        scratch_types=[
            pltpu.VMEM((N,), _F32),
            pltpu.VMEM((N,), _F32),
            pltpu.VMEM((per_w,), jnp.int32),
            pltpu.VMEM((per_w,), jnp.int32),
            pltpu.VMEM((per_w,), _F32),
            pltpu.VMEM((per_w,), _F32),
        ],
    )
    def k(px_hbm, py_hbm, dst_hbm, src_hbm, dx_hbm, dy_hbm,
          px_v, py_v, id_v, is_v, dx_v, dy_v):
        wid = jax.lax.axis_index("s") * NC + jax.lax.axis_index("c")
        base = wid * per_w
        pltpu.sync_copy(px_hbm, px_v)
        pltpu.sync_copy(py_hbm, py_v)
        pltpu.sync_copy(dst_hbm.at[pl.ds(base, per_w)], id_v)
        pltpu.sync_copy(src_hbm.at[pl.ds(base, per_w)], is_v)

        def body(kk):
            d = id_v[pl.ds(kk, 16)]
            s = is_v[pl.ds(kk, 16)]
            dx_v[pl.ds(kk, 16)] = (plsc.load_gather(px_v, [d])
                                   - plsc.load_gather(px_v, [s]))
            dy_v[pl.ds(kk, 16)] = (plsc.load_gather(py_v, [d])
                                   - plsc.load_gather(py_v, [s]))

        nfull = per_w // 16 * 16

        @pl.loop(0, nfull, step=16)
        def _(kk):
            body(kk)

        if per_w % 16:
            body(per_w - 16)

        pltpu.sync_copy(dx_v, dx_hbm.at[pl.ds(base, per_w)])
        pltpu.sync_copy(dy_v, dy_hbm.at[pl.ds(base, per_w)])

    return k(px, py, dst, src)


def _sc_scatter_add(u, p128, dst, zu):
    c = 200
    cs = 40
    nsb = c // cs
    ee = dst.shape[0]
    ng = ee // c
    nbase, nrem = divmod(ng, NS)
    rps = NPAD // NS
    mesh = plsc.VectorSubcoreMesh(core_axis_name="c", subcore_axis_name="s")

    @functools.partial(
        pl.kernel,
        mesh=mesh,
        out_type=[
            jax.ShapeDtypeStruct((NPAD, H), _F32),
            jax.ShapeDtypeStruct((NPAD, H), _F32),
        ],
        scratch_types=[
            pltpu.VMEM((c,), jnp.int32),
            pltpu.VMEM((nsb, cs), jnp.int32),
            pltpu.VMEM((c, H), _F32),
            pltpu.VMEM_SHARED((NPAD, H), _F32),
            pltpu.SemaphoreType.DMA,
        ],
    )
    def k(u_hbm, p_hbm, dst_hbm, zu_hbm, ou_hbm, op_hbm,
          idx_v, idx2_v, d_v, acc, sem):
        ci = jax.lax.axis_index("c")
        si = jax.lax.axis_index("s")
        row0 = si * rps
        pltpu.sync_copy(zu_hbm.at[pl.ds(row0, rps)], acc.at[pl.ds(row0, rps)])
        plsc.subcore_barrier()

        def body(src_hbm):
            nch = nbase + jnp.where(si < nrem, 1, 0)

            @pl.loop(0, nch)
            def _(j):
                base = (si + j * NS) * c
                pltpu.sync_copy(dst_hbm.at[pl.ds(base, c)], idx_v)
                pltpu.sync_copy(src_hbm.at[pl.ds(base, c)], d_v)

                for sb in range(nsb):
                    for kk in list(range(0, cs - 15, 16)) + (
                            [cs - 16] if cs % 16 else []):
                        idx2_v[sb, pl.ds(kk, 16)] = (
                            idx_v[pl.ds(sb * cs + kk, 16)])

                cps = [pltpu.async_copy(
                    d_v.at[pl.ds(sb * cs, cs)],
                    acc.at[idx2_v.at[sb]], sem, add=True)
                    for sb in range(nsb)]
                for cp in cps:
                    cp.wait()

        @pl.when(ci == 0)
        def _():
            body(u_hbm)

        @pl.when(ci == 1)
        def _():
            body(p_hbm)

        plsc.subcore_barrier()

        @pl.when(ci == 0)
        def _():
            pltpu.sync_copy(acc.at[pl.ds(row0, rps)],
                            ou_hbm.at[pl.ds(row0, rps)])

        @pl.when(ci == 1)
        def _():
            pltpu.sync_copy(acc.at[pl.ds(row0, rps)],
                            op_hbm.at[pl.ds(row0, rps)])

    return k(u, p128, dst, zu)


def _r1(v):
    return v.reshape(1, -1)


def kernel(field, pos, bc_type, edge_index, face_normals, face_areas,
           face_type, params):
    src = edge_index[0].astype(jnp.int32)
    dst = edge_index[1].astype(jnp.int32)

    lane = jax.lax.broadcasted_iota(jnp.int32, (H, 16), 0) // 16
    head = jax.lax.broadcasted_iota(jnp.int32, (H, 16), 1)
    sel = jnp.where(lane == head, 0.25, 0.0).astype(_F32)
    exp16 = sel.T * 4.0

    ne = params['node_enc']
    dist_lo = jnp.clip(pos, -1.0, 1.0)
    dist_hi = jnp.clip(1.0 - pos, -1.0, 1.0)
    bcoh = (bc_type[:, None] == jnp.arange(5)[None, :]).astype(_F32)
    nf16 = jnp.concatenate(
        [field, dist_lo, dist_hi, bcoh, jnp.zeros((N, 3), _F32)], axis=1)
    wn1 = ne['linears'][0]['W']
    wn16 = jnp.concatenate(
        [wn1[:8], params['bc_embed'] @ wn1[8:16], jnp.zeros((3, H), _F32)], axis=0)
    x = _mlp16(nf16, wn16, _r1(ne['linears'][0]['b']),
               ne['linears'][1]['W'], _r1(ne['linears'][1]['b']),
               ne['linears'][2]['W'], _r1(ne['linears'][2]['b']),
               _r1(ne['ln']['g']), _r1(ne['ln']['b']))

    dx, dy = _sc_pos_delta(pos[:, 0], pos[:, 1], dst, src)
    delta = jnp.stack([dx, dy], axis=1)
    dist = jnp.maximum(
        jnp.sqrt(jnp.sum(delta * delta, axis=1, keepdims=True)), 1e-8)
    unit = delta / dist
    ftoh = (face_type[:, None] == jnp.arange(4)[None, :]).astype(_F32)
    ef16 = jnp.concatenate(
        [face_normals, face_areas[:, None], dist, unit, ftoh,
         jnp.zeros((E, 6), _F32)], axis=1)
    ee = params['edge_enc']
    we1 = ee['linears'][0]['W']
    we16 = jnp.concatenate(
        [we1[:6], params['ft_embed'] @ we1[6:10], jnp.zeros((6, H), _F32)], axis=0)
    e = _mlp16(ef16, we16, _r1(ee['linears'][0]['b']),
               ee['linears'][1]['W'], _r1(ee['linears'][1]['b']),
               ee['linears'][2]['W'], _r1(ee['linears'][2]['b']),
               _r1(ee['ln']['g']), _r1(ee['ln']['b']))

    zu = jnp.zeros((NPAD, H), _F32)
    expb128 = jnp.concatenate([exp16, jnp.zeros((H - 16, H), _F32)], axis=0)
    dec = params['dec']
    dec_ws = (dec['linears'][0]['W'], _r1(dec['linears'][0]['b']),
              dec['linears'][1]['W'], _r1(dec['linears'][1]['b']),
              jnp.pad(dec['linears'][2]['W'], ((0, 0), (0, H - 4))),
              _r1(jnp.pad(dec['linears'][2]['b'], (0, H - 4))))

    ec = E // 2
    dst_h = (dst[:ec], dst[ec:])
    src_h = (src[:ec], src[ec:])
    e_h = [e[:ec], e[ec:]]

    for b, bp in enumerate(params['blocks']):
        em = bp['edge_mlp']
        wqk = jnp.block([[bp['W_Q'], jnp.zeros((H, H), _F32)],
                         [jnp.zeros((H, H), _F32), bp['W_K']]])
        emit_e = b == 0
        accs = []
        e_next_h = []
        for hh in range(2):
            gxd, gxs = _sc_gather2(x, dst_h[hh], src_h[hh])
            eouts = _edge_block(
                emit_e, gxd, gxs, e_h[hh],
                em['linears'][0]['W'], _r1(em['linears'][0]['b']),
                em['linears'][1]['W'], _r1(em['linears'][1]['b']),
                em['linears'][2]['W'], _r1(em['linears'][2]['b']),
                _r1(em['ln']['g']), _r1(em['ln']['b']),
                wqk, bp['W_V'], sel, exp16)
            if emit_e:
                e_next, u, p128 = eouts
                e_next_h.append(e_next)
            else:
                u, p128 = eouts
            accs.extend(_sc_scatter_add(u, p128, dst_h[hh], zu))
        nm = bp['node_mlp']
        node_ws = (nm['linears'][0]['W'], _r1(nm['linears'][0]['b']),
                   nm['linears'][1]['W'], _r1(nm['linears'][1]['b']),
                   nm['linears'][2]['W'], _r1(nm['linears'][2]['b']),
                   _r1(nm['ln']['g']), _r1(nm['ln']['b']),
                   bp['W_O'], expb128)
        x = _node_block(x, accs, node_ws,
                        dec_weights=dec_ws if b == 1 else None)
        if emit_e:
            e_h = e_next_h

    return x[:, :4]


# speedup vs baseline: 6.1191x; 1.0440x over previous
import dataclasses
import functools

import jax
import jax.numpy as jnp
from jax.experimental import pallas as pl
from jax.experimental.pallas import tpu as pltpu
from jax.experimental.pallas import tpu_sc as plsc

N = 10000
E = 160000
H = 128
NHEADS = 8
NPAD = 10240
NC, NS = 2, 16
NW = NC * NS
TT = 2000

_F32 = jnp.float32


def _ln(h, g, b):
    mu = jnp.mean(h, axis=1, keepdims=True)
    var = jnp.mean((h - mu) ** 2, axis=1, keepdims=True)
    return (h - mu) * jax.lax.rsqrt(var + 1e-5) * g + b


def _dot(a, b):
    return jnp.dot(a, b, preferred_element_type=_F32)


def _enc_body(xin, w1, b1, w2, b2, w3, b3, g, bt, out):
    h = jnp.maximum(_dot(xin[...], w1[...]) + b1[...], 0.0)
    h = jnp.maximum(_dot(h, w2[...]) + b2[...], 0.0)
    h = _dot(h, w3[...]) + b3[...]
    out[...] = _ln(h, g[...], bt[...])


def _mlp16(xin, w1, b1, w2, b2, w3, b3, g, bt):
    rows = xin.shape[0]
    grid = rows // TT
    wspec = lambda a: pl.BlockSpec(a.shape, lambda i: (0,) * a.ndim)
    return pl.pallas_call(
        _enc_body,
        grid=(grid,),
        in_specs=[pl.BlockSpec((TT, 16), lambda i: (i, 0))]
        + [wspec(a) for a in (w1, b1, w2, b2, w3, b3, g, bt)],
        out_specs=pl.BlockSpec((TT, H), lambda i: (i, 0)),
        out_shape=jax.ShapeDtypeStruct((rows, H), _F32),
    )(xin, w1, b1, w2, b2, w3, b3, g, bt)


def _edge_body(emit_e, gxd, gxs, e, w1, b1, w2, b2, w3, b3, g, bt, wqk, wv,
               sel, exp16, *outs):
    if emit_e:
        e1_out, u_out, p_out = outs
    else:
        u_out, p_out = outs
    xd = gxd[...]
    xs = gxs[...]
    ev = e[...]
    cat = jnp.concatenate([xd, xs, ev], axis=1)
    h = jnp.maximum(_dot(cat, w1[...]) + b1[...], 0.0)
    h = jnp.maximum(_dot(h, w2[...]) + b2[...], 0.0)
    h = _dot(h, w3[...]) + b3[...]
    e_new = _ln(h, g[...], bt[...])
    if emit_e:
        e1_out[...] = ev + e_new
    qk = _dot(jnp.concatenate([xd, xs], axis=1), wqk[...])
    prod = qk[:, :H] * qk[:, H:]
    p = jnp.exp(_dot(prod, sel[...]))
    v = _dot(e_new, wv[...])
    u_out[...] = _dot(p, exp16[...]) * v
    p_out[...] = jnp.concatenate(
        [p, jnp.zeros((p.shape[0], H - 16), _F32)], axis=1)


def _edge_block(emit_e, gxd, gxs, e, w1, b1, w2, b2, w3, b3, g, bt, wqk, wv,
                sel, exp16):
    ee = gxd.shape[0]
    grid = ee // TT
    wspec = lambda a: pl.BlockSpec(a.shape, lambda i: (0,) * a.ndim)
    ws = (w1, b1, w2, b2, w3, b3, g, bt, wqk, wv, sel, exp16)
    nout = 3 if emit_e else 2
    return pl.pallas_call(
        functools.partial(_edge_body, emit_e),
        grid=(grid,),
        in_specs=[pl.BlockSpec((TT, H), lambda i: (i, 0))] * 3
        + [wspec(a) for a in ws],
        out_specs=[pl.BlockSpec((TT, H), lambda i: (i, 0))] * nout,
        out_shape=[jax.ShapeDtypeStruct((ee, H), _F32)] * nout,
    )(gxd, gxs, e, *ws)


def _node_body(dec, x, accu1, accp1, accu2, accp2,
               w1, b1, w2, b2, w3, b3, g, bt, wo, expb, *rest):
    if dec:
        (wd1, bd1, wd2, bd2, wd3, bd3, out) = rest
    else:
        (out,) = rest
    xv = x[...]
    u = accu1[...] + accu2[...]
    pm = accp1[...] + accp2[...]
    rec = 1.0 / (pm + 1e-12)
    recb = _dot(rec, expb[...])
    msg = _dot(u * recb, wo[...])
    cat = jnp.concatenate([xv, msg], axis=1)
    h = jnp.maximum(_dot(cat, w1[...]) + b1[...], 0.0)
    h = jnp.maximum(_dot(h, w2[...]) + b2[...], 0.0)
    h = _dot(h, w3[...]) + b3[...]
    xn = xv + _ln(h, g[...], bt[...])
    if dec:
        h = jnp.maximum(_dot(xn, wd1[...]) + bd1[...], 0.0)
        h = jnp.maximum(_dot(h, wd2[...]) + bd2[...], 0.0)
        out[...] = _dot(h, wd3[...]) + bd3[...]
    else:
        out[...] = xn


def _node_block(x, accs, weights, dec_weights=None):
    grid = N // TT
    wspec = lambda a: pl.BlockSpec(a.shape, lambda i: (0,) * a.ndim)
    dec = dec_weights is not None
    ws = tuple(weights) + (tuple(dec_weights) if dec else ())
    return pl.pallas_call(
        functools.partial(_node_body, dec),
        grid=(grid,),
        in_specs=[pl.BlockSpec((TT, H), lambda i: (i, 0))] * 5
        + [wspec(a) for a in ws],
        out_specs=pl.BlockSpec((TT, H), lambda i: (i, 0)),
        out_shape=jax.ShapeDtypeStruct((N, H), _F32),
    )(x, *accs, *ws)


def _sc_gather2(table, idx_a, idx_b):
    d = table.shape[1]
    ee = idx_a.shape[0]
    ci = 400
    ng = ee // ci
    nbase, nrem = divmod(ng, NW)
    mesh = plsc.VectorSubcoreMesh(core_axis_name="c", subcore_axis_name="s")

    @functools.partial(
        pl.kernel,
        mesh=mesh,
        out_type=[jax.ShapeDtypeStruct((ee, d), _F32)] * 2,
        scratch_types=[
            pltpu.VMEM((ci,), jnp.int32),
            pltpu.VMEM((ci,), jnp.int32),
            pltpu.VMEM((ci, d), _F32),
            pltpu.VMEM((ci, d), _F32),
            pltpu.SemaphoreType.DMA,
            pltpu.SemaphoreType.DMA,
        ],
    )
    def k(tab_hbm, ia_hbm, ib_hbm, oa_hbm, ob_hbm,
          ia_v, ib_v, ra_v, rb_v, semg, semw):
        wid = jax.lax.axis_index("s") * NC + jax.lax.axis_index("c")
        nch = nbase + jnp.where(wid < nrem, 1, 0)

        @pl.loop(0, nch)
        def _(j):
            base = (wid + j * NW) * ci
            cia = pltpu.async_copy(ia_hbm.at[pl.ds(base, ci)], ia_v, semg)
            cib = pltpu.async_copy(ib_hbm.at[pl.ds(base, ci)], ib_v, semg)
            cia.wait()
            cib.wait()

            @pl.when(j > 0)
            def _():
                pbase = (wid + (j - 1) * NW) * ci
                pltpu.make_async_copy(
                    ra_v, oa_hbm.at[pl.ds(pbase, ci)], semw).wait()
                pltpu.make_async_copy(
                    rb_v, ob_hbm.at[pl.ds(pbase, ci)], semw).wait()

            cga = pltpu.async_copy(tab_hbm.at[ia_v], ra_v, semg)
            cgb = pltpu.async_copy(tab_hbm.at[ib_v], rb_v, semg)
            cga.wait()
            cgb.wait()
            pltpu.async_copy(ra_v, oa_hbm.at[pl.ds(base, ci)], semw)
            pltpu.async_copy(rb_v, ob_hbm.at[pl.ds(base, ci)], semw)

        lbase = (wid + (nch - 1) * NW) * ci
        pltpu.make_async_copy(ra_v, oa_hbm.at[pl.ds(lbase, ci)], semw).wait()
        pltpu.make_async_copy(rb_v, ob_hbm.at[pl.ds(lbase, ci)], semw).wait()

    return k(table, idx_a, idx_b)


def _sc_pos_delta(px, py, dst, src):
    per_w = E // NW
    mesh = plsc.VectorSubcoreMesh(core_axis_name="c", subcore_axis_name="s")
    cp = pltpu.CompilerParams()
    if "needs_layout_passes" in pltpu.CompilerParams.__dataclass_fields__:
        cp = dataclasses.replace(cp, needs_layout_passes=False)

    @functools.partial(
        pl.kernel,
        mesh=mesh,
        compiler_params=cp,
        out_type=[jax.ShapeDtypeStruct((E,), _F32)] * 2,
        scratch_types=[
            pltpu.VMEM((N,), _F32),
            pltpu.VMEM((N,), _F32),
            pltpu.VMEM((per_w,), jnp.int32),
            pltpu.VMEM((per_w,), jnp.int32),
            pltpu.VMEM((per_w,), _F32),
            pltpu.VMEM((per_w,), _F32),
        ],
    )
    def k(px_hbm, py_hbm, dst_hbm, src_hbm, dx_hbm, dy_hbm,
          px_v, py_v, id_v, is_v, dx_v, dy_v):
        wid = jax.lax.axis_index("s") * NC + jax.lax.axis_index("c")
        base = wid * per_w
        pltpu.sync_copy(px_hbm, px_v)
        pltpu.sync_copy(py_hbm, py_v)
        pltpu.sync_copy(dst_hbm.at[pl.ds(base, per_w)], id_v)
        pltpu.sync_copy(src_hbm.at[pl.ds(base, per_w)], is_v)

        def body(kk):
            d = id_v[pl.ds(kk, 16)]
            s = is_v[pl.ds(kk, 16)]
            dx_v[pl.ds(kk, 16)] = (plsc.load_gather(px_v, [d])
                                   - plsc.load_gather(px_v, [s]))
            dy_v[pl.ds(kk, 16)] = (plsc.load_gather(py_v, [d])
                                   - plsc.load_gather(py_v, [s]))

        nfull = per_w // 16 * 16

        @pl.loop(0, nfull, step=16)
        def _(kk):
            body(kk)

        if per_w % 16:
            body(per_w - 16)

        pltpu.sync_copy(dx_v, dx_hbm.at[pl.ds(base, per_w)])
        pltpu.sync_copy(dy_v, dy_hbm.at[pl.ds(base, per_w)])

    return k(px, py, dst, src)


def _sc_scatter_add(u, p128, dst, zu):
    c = 200
    cs = 40
    nsb = c // cs
    ee = dst.shape[0]
    ng = ee // c
    nbase, nrem = divmod(ng, NS)
    rps = NPAD // NS
    mesh = plsc.VectorSubcoreMesh(core_axis_name="c", subcore_axis_name="s")

    @functools.partial(
        pl.kernel,
        mesh=mesh,
        out_type=[
            jax.ShapeDtypeStruct((NPAD, H), _F32),
            jax.ShapeDtypeStruct((NPAD, H), _F32),
        ],
        scratch_types=[
            pltpu.VMEM((c,), jnp.int32),
            pltpu.VMEM((nsb, cs), jnp.int32),
            pltpu.VMEM((c, H), _F32),
            pltpu.VMEM_SHARED((NPAD, H), _F32),
            pltpu.SemaphoreType.DMA,
            pltpu.SemaphoreType.DMA,
        ],
    )
    def k(u_hbm, p_hbm, dst_hbm, zu_hbm, ou_hbm, op_hbm,
          idx_v, idx2_v, d_v, acc, sem, seml):
        ci = jax.lax.axis_index("c")
        si = jax.lax.axis_index("s")
        row0 = si * rps
        pltpu.sync_copy(zu_hbm.at[pl.ds(row0, rps)], acc.at[pl.ds(row0, rps)])
        plsc.subcore_barrier()

        def body(src_hbm):
            nch = nbase + jnp.where(si < nrem, 1, 0)

            @pl.loop(0, nch)
            def _(j):
                base = (si + j * NS) * c
                pltpu.sync_copy(dst_hbm.at[pl.ds(base, c)], idx_v)
                loads = [pltpu.async_copy(
                    src_hbm.at[pl.ds(base + sb * cs, cs)],
                    d_v.at[pl.ds(sb * cs, cs)], seml)
                    for sb in range(nsb)]

                for sb in range(nsb):
                    for kk in list(range(0, cs - 15, 16)) + (
                            [cs - 16] if cs % 16 else []):
                        idx2_v[sb, pl.ds(kk, 16)] = (
                            idx_v[pl.ds(sb * cs + kk, 16)])

                cps = []
                for sb in range(nsb):
                    loads[sb].wait()
                    cps.append(pltpu.async_copy(
                        d_v.at[pl.ds(sb * cs, cs)],
                        acc.at[idx2_v.at[sb]], sem, add=True))
                for cp in cps:
                    cp.wait()

        @pl.when(ci == 0)
        def _():
            body(u_hbm)

        @pl.when(ci == 1)
        def _():
            body(p_hbm)

        plsc.subcore_barrier()

        @pl.when(ci == 0)
        def _():
            pltpu.sync_copy(acc.at[pl.ds(row0, rps)],
                            ou_hbm.at[pl.ds(row0, rps)])

        @pl.when(ci == 1)
        def _():
            pltpu.sync_copy(acc.at[pl.ds(row0, rps)],
                            op_hbm.at[pl.ds(row0, rps)])

    return k(u, p128, dst, zu)


def _r1(v):
    return v.reshape(1, -1)


def kernel(field, pos, bc_type, edge_index, face_normals, face_areas,
           face_type, params):
    src = edge_index[0].astype(jnp.int32)
    dst = edge_index[1].astype(jnp.int32)

    lane = jax.lax.broadcasted_iota(jnp.int32, (H, 16), 0) // 16
    head = jax.lax.broadcasted_iota(jnp.int32, (H, 16), 1)
    sel = jnp.where(lane == head, 0.25, 0.0).astype(_F32)
    exp16 = sel.T * 4.0

    ne = params['node_enc']
    dist_lo = jnp.clip(pos, -1.0, 1.0)
    dist_hi = jnp.clip(1.0 - pos, -1.0, 1.0)
    bcoh = (bc_type[:, None] == jnp.arange(5)[None, :]).astype(_F32)
    nf16 = jnp.concatenate(
        [field, dist_lo, dist_hi, bcoh, jnp.zeros((N, 3), _F32)], axis=1)
    wn1 = ne['linears'][0]['W']
    wn16 = jnp.concatenate(
        [wn1[:8], params['bc_embed'] @ wn1[8:16], jnp.zeros((3, H), _F32)], axis=0)
    x = _mlp16(nf16, wn16, _r1(ne['linears'][0]['b']),
               ne['linears'][1]['W'], _r1(ne['linears'][1]['b']),
               ne['linears'][2]['W'], _r1(ne['linears'][2]['b']),
               _r1(ne['ln']['g']), _r1(ne['ln']['b']))

    dx, dy = _sc_pos_delta(pos[:, 0], pos[:, 1], dst, src)
    delta = jnp.stack([dx, dy], axis=1)
    dist = jnp.maximum(
        jnp.sqrt(jnp.sum(delta * delta, axis=1, keepdims=True)), 1e-8)
    unit = delta / dist
    ftoh = (face_type[:, None] == jnp.arange(4)[None, :]).astype(_F32)
    ef16 = jnp.concatenate(
        [face_normals, face_areas[:, None], dist, unit, ftoh,
         jnp.zeros((E, 6), _F32)], axis=1)
    ee = params['edge_enc']
    we1 = ee['linears'][0]['W']
    we16 = jnp.concatenate(
        [we1[:6], params['ft_embed'] @ we1[6:10], jnp.zeros((6, H), _F32)], axis=0)
    e = _mlp16(ef16, we16, _r1(ee['linears'][0]['b']),
               ee['linears'][1]['W'], _r1(ee['linears'][1]['b']),
               ee['linears'][2]['W'], _r1(ee['linears'][2]['b']),
               _r1(ee['ln']['g']), _r1(ee['ln']['b']))

    zu = jnp.zeros((NPAD, H), _F32)
    expb128 = jnp.concatenate([exp16, jnp.zeros((H - 16, H), _F32)], axis=0)
    dec = params['dec']
    dec_ws = (dec['linears'][0]['W'], _r1(dec['linears'][0]['b']),
              dec['linears'][1]['W'], _r1(dec['linears'][1]['b']),
              jnp.pad(dec['linears'][2]['W'], ((0, 0), (0, H - 4))),
              _r1(jnp.pad(dec['linears'][2]['b'], (0, H - 4))))

    ec = E // 2
    dst_h = (dst[:ec], dst[ec:])
    src_h = (src[:ec], src[ec:])
    e_h = [e[:ec], e[ec:]]

    for b, bp in enumerate(params['blocks']):
        em = bp['edge_mlp']
        wqk = jnp.block([[bp['W_Q'], jnp.zeros((H, H), _F32)],
                         [jnp.zeros((H, H), _F32), bp['W_K']]])
        emit_e = b == 0
        accs = []
        e_next_h = []
        for hh in range(2):
            gxd, gxs = _sc_gather2(x, dst_h[hh], src_h[hh])
            eouts = _edge_block(
                emit_e, gxd, gxs, e_h[hh],
                em['linears'][0]['W'], _r1(em['linears'][0]['b']),
                em['linears'][1]['W'], _r1(em['linears'][1]['b']),
                em['linears'][2]['W'], _r1(em['linears'][2]['b']),
                _r1(em['ln']['g']), _r1(em['ln']['b']),
                wqk, bp['W_V'], sel, exp16)
            if emit_e:
                e_next, u, p128 = eouts
                e_next_h.append(e_next)
            else:
                u, p128 = eouts
            accs.extend(_sc_scatter_add(u, p128, dst_h[hh], zu))
        nm = bp['node_mlp']
        node_ws = (nm['linears'][0]['W'], _r1(nm['linears'][0]['b']),
                   nm['linears'][1]['W'], _r1(nm['linears'][1]['b']),
                   nm['linears'][2]['W'], _r1(nm['linears'][2]['b']),
                   _r1(nm['ln']['g']), _r1(nm['ln']['b']),
                   bp['W_O'], expb128)
        x = _node_block(x, accs, node_ws,
                        dec_weights=dec_ws if b == 1 else None)
        if emit_e:
            e_h = e_next_h

    return x[:, :4]
